# Initial kernel scaffold; baseline (speedup 1.0000x reference)
import jax
import jax.numpy as jnp
from jax.experimental import pallas as pl


def kernel(gate_types, edge_index, physical_idx, gate_embed, W1_0, W2_0, W1_1, W2_1, dev_table, Wq, Wk, Wv, Wo, ln1_g, ln1_b, ffn_W1, ffn_b1, ffn_W2, ffn_b2, ln2_g, ln2_b):
    raise NotImplementedError("write your pallas kernel here")


# speedup vs baseline: 7.1601x; 7.1601x over previous
import functools

import jax
import jax.numpy as jnp
from jax import lax
from jax.experimental import pallas as pl
from jax.experimental.pallas import tpu as pltpu
from jax.experimental.pallas import tpu_sc as plsc

B = 500
G = 100
Q = 32
R = 64
N = B * G
E = 800000
NUM_GATE_TYPES = 32
EMB = 64
INTER = 64
HF = 64
DEV = 64
DM = HF + DEV
NH = 4
DK = 32
DV = 32
DI = DM * 2

NC = 2
NS = 16

EDGE_CHUNK = 128
SUB = 4
BLK_E = EDGE_CHUNK * SUB
BLOCKS_PER_TILE = 100
E_TILE = BLK_E * BLOCKS_PER_TILE
E_PAD = E_TILE * NS
N_SP = 51200
ROWS_TILE = N_SP // NS
ZROWS = 160

BN = 2000
BB = 16
BP = 512


def _stage_a_body(gt_ref, ge_ref, w1_ref, out_ref):
    tbl = jnp.maximum(ge_ref[...] @ w1_ref[...], 0.0)
    gt = gt_ref[0]
    io = lax.broadcasted_iota(jnp.int32, (NUM_GATE_TYPES, BN), 0)
    oh = (io == gt).astype(jnp.float32)
    t = lax.dot_general(oh, tbl, (((0,), (0,)), ((), ())))
    out_ref[0] = t[:, :32]
    out_ref[1] = t[:, 32:]


def _stage_a(gate_types, gate_embed, W1_0):
    gt3 = gate_types.astype(jnp.int32).reshape(N // BN, 1, BN)
    return pl.pallas_call(
        _stage_a_body,
        grid=(N // BN,),
        in_specs=[
            pl.BlockSpec((1, 1, BN), lambda i: (i, 0, 0)),
            pl.BlockSpec((NUM_GATE_TYPES, EMB), lambda i: (0, 0)),
            pl.BlockSpec((EMB, INTER), lambda i: (0, 0)),
        ],
        out_specs=pl.BlockSpec((2, BN, 32), lambda i: (0, i, 0)),
        out_shape=jax.ShapeDtypeStruct((2, N, 32), jnp.float32),
    )(gt3, gate_embed, W1_0)


def _sc_body(tbl_hbm, src_hbm, dst_hbm, out_hbm,
             agg_sp, zbuf, rows, srcv, dstv, sem):
    c = lax.axis_index("c")
    s = lax.axis_index("s")

    def _z(i, carry):
        zbuf[i, pl.ds(0, 16)] = jnp.zeros((16,), jnp.float32)
        zbuf[i, pl.ds(16, 16)] = jnp.zeros((16,), jnp.float32)
        return carry
    lax.fori_loop(0, ZROWS, _z, 0)
    base_r = s * ROWS_TILE
    for k in range(ROWS_TILE // ZROWS):
        pltpu.sync_copy(zbuf, agg_sp.at[pl.ds(base_r + k * ZROWS, ZROWS)])
    plsc.subcore_barrier()

    def _blk(b, carry):
        g = s * BLOCKS_PER_TILE + b
        pltpu.sync_copy(src_hbm.at[c, g], srcv)
        pltpu.sync_copy(dst_hbm.at[g], dstv)
        cps = [pltpu.async_copy(tbl_hbm.at[srcv.at[j]], rows.at[j], sem)
               for j in range(SUB)]
        for cp in cps:
            cp.wait()
        for j in range(SUB):
            pltpu.sync_copy(rows.at[j], agg_sp.at[dstv.at[j]], add=True)
        return carry
    lax.fori_loop(0, BLOCKS_PER_TILE, _blk, 0)
    plsc.subcore_barrier()

    for k in range(ROWS_TILE // ZROWS):
        r0 = base_r + k * ZROWS
        pltpu.sync_copy(agg_sp.at[pl.ds(r0, ZROWS)], zbuf)
        pltpu.sync_copy(zbuf, out_hbm.at[c, pl.ds(r0, ZROWS)])


def _make_sc_call():
    mesh = plsc.VectorSubcoreMesh(core_axis_name="c", subcore_axis_name="s")
    return pl.kernel(
        _sc_body,
        out_type=jax.ShapeDtypeStruct((2, N_SP, 32), jnp.float32),
        mesh=mesh,
        scratch_types=[
            pltpu.VMEM_SHARED((N_SP, 32), jnp.float32),
            pltpu.VMEM((ZROWS, 32), jnp.float32),
            pltpu.VMEM((SUB, EDGE_CHUNK, 32), jnp.float32),
            pltpu.VMEM((SUB, EDGE_CHUNK), jnp.int32),
            pltpu.VMEM((SUB, EDGE_CHUNK), jnp.int32),
            pltpu.SemaphoreType.DMA,
        ],
        compiler_params=pltpu.CompilerParams(use_tc_tiling_on_sc=False),
    )


def _stage_b_body(agg_ref, w2_ref, w1_ref, out_ref):
    agg = jnp.concatenate([agg_ref[0], agg_ref[1]], axis=1)
    h = jnp.maximum(agg @ w2_ref[...], 0.0)
    t = jnp.maximum(h @ w1_ref[...], 0.0)
    out_ref[0] = t[:, :32]
    out_ref[1] = t[:, 32:]


def _stage_b(agg, W2_0, W1_1):
    return pl.pallas_call(
        _stage_b_body,
        grid=(N // BN,),
        in_specs=[
            pl.BlockSpec((2, BN, 32), lambda i: (0, i, 0)),
            pl.BlockSpec((INTER, HF), lambda i: (0, 0)),
            pl.BlockSpec((HF, INTER), lambda i: (0, 0)),
        ],
        out_specs=pl.BlockSpec((2, BN, 32), lambda i: (0, i, 0)),
        out_shape=jax.ShapeDtypeStruct((2, N, 32), jnp.float32),
    )(agg, W2_0, W1_1)


def _stage_c_body(agg_ref, phys_ref, w2_ref, dev_ref, wq_ref, wk_ref,
                  wv_ref, wo_ref, ln1g_ref, ln1b_ref, fw1_ref, fb1_ref,
                  fw2_ref, fb2_ref, ln2g_ref, ln2b_ref, out_ref, asum_ref):
    agg = jnp.concatenate([agg_ref[0], agg_ref[1]], axis=1)
    aggq = agg.reshape(BB, G, HF)[:, :Q, :].reshape(BB * Q, HF)
    hq = jnp.maximum(aggq @ w2_ref[...], 0.0).reshape(BB, Q, HF)

    phys_q = phys_ref[...][:, :Q]
    io_r = lax.broadcasted_iota(jnp.int32, (BB, R, Q), 1)
    mask = (phys_q[:, None, :] == io_r).astype(jnp.float32)
    x_rep = lax.dot_general(mask, hq, (((2,), (1,)), ((0,), (0,))))

    dev = jnp.broadcast_to(dev_ref[...][None], (BB, R, DEV))
    x = jnp.concatenate([x_rep, dev], axis=2)
    xf = x.reshape(BB * R, DM)

    q = xf @ wq_ref[...]
    k = xf @ wk_ref[...]
    v = xf @ wv_ref[...]
    scale = 1.0 / (DK ** 0.5)

    mha = jnp.zeros((BB * R, DM), jnp.float32)
    asum = jnp.zeros((BB, R, R), jnp.float32)
    wo = wo_ref[...]
    for h in range(NH):
        qh = q[:, h * DK:(h + 1) * DK].reshape(BB, R, DK)
        kh = k[:, h * DK:(h + 1) * DK].reshape(BB, R, DK)
        vh = v[:, h * DV:(h + 1) * DV].reshape(BB, R, DV)
        s = lax.dot_general(qh, kh, (((2,), (2,)), ((0,), (0,)))) * scale
        s = s - jnp.max(s, axis=2, keepdims=True)
        es = jnp.exp(s)
        attn = es / jnp.sum(es, axis=2, keepdims=True)
        ctx = lax.dot_general(attn, vh, (((2,), (1,)), ((0,), (0,))))
        mha = mha + ctx.reshape(BB * R, DV) @ wo[h * DV:(h + 1) * DV, :]
        asum = asum + attn

    def _ln(t, g, b):
        m = jnp.mean(t, axis=1, keepdims=True)
        d = t - m
        var = jnp.mean(d * d, axis=1, keepdims=True)
        return d * lax.rsqrt(var + 1e-6) * g + b

    res = _ln(xf + mha, ln1g_ref[...], ln1b_ref[...])
    ffn = jnp.maximum(res @ fw1_ref[...] + fb1_ref[...], 0.0)
    ffn = ffn @ fw2_ref[...] + fb2_ref[...]
    out = _ln(res + ffn, ln2g_ref[...], ln2b_ref[...])

    out_ref[...] = out.reshape(BB, R, DM)
    asum_ref[...] = asum


def _stage_c(agg2, phys_p, W2_1, dev_table, Wq, Wk, Wv, Wo,
             ln1_g, ln1_b, ffn_W1, ffn_b1, ffn_W2, ffn_b2, ln2_g, ln2_b):
    full = lambda shape: pl.BlockSpec(shape, lambda i: tuple(0 for _ in shape))
    return pl.pallas_call(
        _stage_c_body,
        grid=(BP // BB,),
        in_specs=[
            pl.BlockSpec((2, BB * G, 32), lambda i: (0, i, 0)),
            pl.BlockSpec((BB, R), lambda i: (i, 0)),
            full((INTER, HF)),
            full((R, DEV)),
            full((DM, NH * DK)),
            full((DM, NH * DK)),
            full((DM, NH * DV)),
            full((NH * DV, DM)),
            full((1, DM)), full((1, DM)),
            full((DM, DI)), full((1, DI)),
            full((DI, DM)), full((1, DM)),
            full((1, DM)), full((1, DM)),
        ],
        out_specs=[
            pl.BlockSpec((BB, R, DM), lambda i: (i, 0, 0)),
            pl.BlockSpec((BB, R, R), lambda i: (i, 0, 0)),
        ],
        out_shape=[
            jax.ShapeDtypeStruct((BP, R, DM), jnp.float32),
            jax.ShapeDtypeStruct((BP, R, R), jnp.float32),
        ],
    )(agg2, phys_p, W2_1, dev_table, Wq, Wk, Wv, Wo,
      ln1_g.reshape(1, DM), ln1_b.reshape(1, DM),
      ffn_W1, ffn_b1.reshape(1, DI), ffn_W2, ffn_b2.reshape(1, DM),
      ln2_g.reshape(1, DM), ln2_b.reshape(1, DM))


def kernel(gate_types, edge_index, physical_idx, gate_embed, W1_0, W2_0,
           W1_1, W2_1, dev_table, Wq, Wk, Wv, Wo, ln1_g, ln1_b, ffn_W1,
           ffn_b1, ffn_W2, ffn_b2, ln2_g, ln2_b):
    src = edge_index[0].astype(jnp.int32)
    dst = edge_index[1].astype(jnp.int32)

    pad = E_PAD - E
    ar = jnp.arange(pad, dtype=jnp.int32)
    src_p = jnp.concatenate([src, (ar * 131) % N])
    dst_p = jnp.concatenate([dst, N + (ar % (N_SP - N))])
    src_both = jnp.stack([src_p, src_p + N]).reshape(
        2, NS * BLOCKS_PER_TILE, SUB, EDGE_CHUNK)
    dst_b = dst_p.reshape(NS * BLOCKS_PER_TILE, SUB, EDGE_CHUNK)

    sc_call = _make_sc_call()

    t0 = _stage_a(gate_types, gate_embed, W1_0)
    agg1 = sc_call(t0.reshape(2 * N, 32), src_both, dst_b)
    t1 = _stage_b(agg1, W2_0, W1_1)
    agg2 = sc_call(t1.reshape(2 * N, 32), src_both, dst_b)

    phys_p = jnp.concatenate(
        [physical_idx.astype(jnp.int32),
         jnp.zeros((BP - B, R), jnp.int32)], axis=0)
    outp, asum = _stage_c(agg2, phys_p, W2_1, dev_table, Wq, Wk, Wv, Wo,
                          ln1_g, ln1_b, ffn_W1, ffn_b1, ffn_W2, ffn_b2,
                          ln2_g, ln2_b)
    return outp[:B], asum[:B]


# speedup vs baseline: 9.2193x; 1.2876x over previous
import functools

import jax
import jax.numpy as jnp
from jax import lax
from jax.experimental import pallas as pl
from jax.experimental.pallas import tpu as pltpu
from jax.experimental.pallas import tpu_sc as plsc

B = 500
G = 100
Q = 32
R = 64
N = B * G
E = 800000
NUM_GATE_TYPES = 32
EMB = 64
INTER = 64
HF = 64
DEV = 64
DM = HF + DEV
NH = 4
DK = 32
DV = 32
DI = DM * 2

NC = 2
NS = 16

EDGE_CHUNK = 128
SUB = 2
BLK_E = EDGE_CHUNK * SUB
SB_BLOCKS = 8
SB_TILE = 25
E_TILE = BLK_E * SB_BLOCKS * SB_TILE
E_PAD = E_TILE * NS
N_SP = 51200
ROWS_TILE = N_SP // NS
ZROWS = 160

BN = 2000
BB = 16
BP = 512


def _stage_a_body(gt_ref, ge_ref, w1_ref, out_ref):
    tbl = jnp.maximum(ge_ref[...] @ w1_ref[...], 0.0)
    gt = gt_ref[0]
    io = lax.broadcasted_iota(jnp.int32, (NUM_GATE_TYPES, BN), 0)
    oh = (io == gt).astype(jnp.float32)
    t = lax.dot_general(oh, tbl, (((0,), (0,)), ((), ())))
    out_ref[0] = t[:, :32]
    out_ref[1] = t[:, 32:]


def _stage_a(gate_types, gate_embed, W1_0):
    gt3 = gate_types.astype(jnp.int32).reshape(N // BN, 1, BN)
    return pl.pallas_call(
        _stage_a_body,
        grid=(N // BN,),
        in_specs=[
            pl.BlockSpec((1, 1, BN), lambda i: (i, 0, 0)),
            pl.BlockSpec((NUM_GATE_TYPES, EMB), lambda i: (0, 0)),
            pl.BlockSpec((EMB, INTER), lambda i: (0, 0)),
        ],
        out_specs=pl.BlockSpec((2, BN, 32), lambda i: (0, i, 0)),
        out_shape=jax.ShapeDtypeStruct((2, N, 32), jnp.float32),
    )(gt3, gate_embed, W1_0)


def _sc_body(tbl_hbm, src_hbm, dst_hbm, out_hbm,
             agg_sp, zbuf, rows0, rows1, srcv, dstv,
             sem_g0, sem_g1, sem_s0, sem_s1):
    c = lax.axis_index("c")
    s = lax.axis_index("s")

    def _z(i, carry):
        zbuf[i, pl.ds(0, 16)] = jnp.zeros((16,), jnp.float32)
        zbuf[i, pl.ds(16, 16)] = jnp.zeros((16,), jnp.float32)
        return carry
    lax.fori_loop(0, ZROWS, _z, 0)
    base_r = s * ROWS_TILE
    for k in range(ROWS_TILE // ZROWS):
        pltpu.sync_copy(zbuf, agg_sp.at[pl.ds(base_r + k * ZROWS, ZROWS)])
    plsc.subcore_barrier()

    rows = (rows0, rows1)
    sem_g = (sem_g0, sem_g1)
    sem_s = (sem_s0, sem_s1)

    def _sb(q, carry):
        sbi = s * SB_TILE + q
        pltpu.sync_copy(src_hbm.at[c, sbi], srcv)
        pltpu.sync_copy(dst_hbm.at[sbi], dstv)

        def fire_g(m):
            rb = rows[m % 2]
            return [pltpu.async_copy(tbl_hbm.at[srcv.at[m, j]], rb.at[j],
                                     sem_g[m % 2]) for j in range(SUB)]

        def fire_s(m):
            rb = rows[m % 2]
            return [pltpu.async_copy(rb.at[j], agg_sp.at[dstv.at[m, j]],
                                     sem_s[m % 2], add=True)
                    for j in range(SUB)]

        g_prev = None
        s_pend = []
        for m in range(SB_BLOCKS):
            if m >= 2:
                for cp in s_pend.pop(0):
                    cp.wait()
            g_cur = fire_g(m)
            if m >= 1:
                for cp in g_prev:
                    cp.wait()
                s_pend.append(fire_s(m - 1))
            g_prev = g_cur
        for cp in g_prev:
            cp.wait()
        s_pend.append(fire_s(SB_BLOCKS - 1))
        for grp in s_pend:
            for cp in grp:
                cp.wait()
        return carry
    lax.fori_loop(0, SB_TILE, _sb, 0)
    plsc.subcore_barrier()

    for k in range(ROWS_TILE // ZROWS):
        r0 = base_r + k * ZROWS
        pltpu.sync_copy(agg_sp.at[pl.ds(r0, ZROWS)], zbuf)
        pltpu.sync_copy(zbuf, out_hbm.at[c, pl.ds(r0, ZROWS)])


def _make_sc_call():
    mesh = plsc.VectorSubcoreMesh(core_axis_name="c", subcore_axis_name="s")
    return pl.kernel(
        _sc_body,
        out_type=jax.ShapeDtypeStruct((2, N_SP, 32), jnp.float32),
        mesh=mesh,
        scratch_types=[
            pltpu.VMEM_SHARED((N_SP, 32), jnp.float32),
            pltpu.VMEM((ZROWS, 32), jnp.float32),
            pltpu.VMEM((SUB, EDGE_CHUNK, 32), jnp.float32),
            pltpu.VMEM((SUB, EDGE_CHUNK, 32), jnp.float32),
            pltpu.VMEM((SB_BLOCKS, SUB, EDGE_CHUNK), jnp.int32),
            pltpu.VMEM((SB_BLOCKS, SUB, EDGE_CHUNK), jnp.int32),
            pltpu.SemaphoreType.DMA,
            pltpu.SemaphoreType.DMA,
            pltpu.SemaphoreType.DMA,
            pltpu.SemaphoreType.DMA,
        ],
        compiler_params=pltpu.CompilerParams(use_tc_tiling_on_sc=False),
    )


def _stage_b_body(agg_ref, w2_ref, w1_ref, out_ref):
    agg = jnp.concatenate([agg_ref[0], agg_ref[1]], axis=1)
    h = jnp.maximum(agg @ w2_ref[...], 0.0)
    t = jnp.maximum(h @ w1_ref[...], 0.0)
    out_ref[0] = t[:, :32]
    out_ref[1] = t[:, 32:]


def _stage_b(agg, W2_0, W1_1):
    return pl.pallas_call(
        _stage_b_body,
        grid=(N // BN,),
        in_specs=[
            pl.BlockSpec((2, BN, 32), lambda i: (0, i, 0)),
            pl.BlockSpec((INTER, HF), lambda i: (0, 0)),
            pl.BlockSpec((HF, INTER), lambda i: (0, 0)),
        ],
        out_specs=pl.BlockSpec((2, BN, 32), lambda i: (0, i, 0)),
        out_shape=jax.ShapeDtypeStruct((2, N, 32), jnp.float32),
    )(agg, W2_0, W1_1)


def _stage_c_body(agg_ref, phys_ref, w2_ref, dev_ref, wq_ref, wk_ref,
                  wv_ref, wo_ref, ln1g_ref, ln1b_ref, fw1_ref, fb1_ref,
                  fw2_ref, fb2_ref, ln2g_ref, ln2b_ref, out_ref, asum_ref):
    agg = jnp.concatenate([agg_ref[0], agg_ref[1]], axis=1)
    aggq = agg.reshape(BB, G, HF)[:, :Q, :].reshape(BB * Q, HF)
    hq = jnp.maximum(aggq @ w2_ref[...], 0.0).reshape(BB, Q, HF)

    phys_q = phys_ref[...][:, :Q]
    io_r = lax.broadcasted_iota(jnp.int32, (BB, R, Q), 1)
    mask = (phys_q[:, None, :] == io_r).astype(jnp.float32)
    x_rep = lax.dot_general(mask, hq, (((2,), (1,)), ((0,), (0,))))

    dev = jnp.broadcast_to(dev_ref[...][None], (BB, R, DEV))
    x = jnp.concatenate([x_rep, dev], axis=2)
    xf = x.reshape(BB * R, DM)

    q = xf @ wq_ref[...]
    k = xf @ wk_ref[...]
    v = xf @ wv_ref[...]
    scale = 1.0 / (DK ** 0.5)

    mha = jnp.zeros((BB * R, DM), jnp.float32)
    asum = jnp.zeros((BB, R, R), jnp.float32)
    wo = wo_ref[...]
    for h in range(NH):
        qh = q[:, h * DK:(h + 1) * DK].reshape(BB, R, DK)
        kh = k[:, h * DK:(h + 1) * DK].reshape(BB, R, DK)
        vh = v[:, h * DV:(h + 1) * DV].reshape(BB, R, DV)
        s = lax.dot_general(qh, kh, (((2,), (2,)), ((0,), (0,)))) * scale
        s = s - jnp.max(s, axis=2, keepdims=True)
        es = jnp.exp(s)
        attn = es / jnp.sum(es, axis=2, keepdims=True)
        ctx = lax.dot_general(attn, vh, (((2,), (1,)), ((0,), (0,))))
        mha = mha + ctx.reshape(BB * R, DV) @ wo[h * DV:(h + 1) * DV, :]
        asum = asum + attn

    def _ln(t, g, b):
        m = jnp.mean(t, axis=1, keepdims=True)
        d = t - m
        var = jnp.mean(d * d, axis=1, keepdims=True)
        return d * lax.rsqrt(var + 1e-6) * g + b

    res = _ln(xf + mha, ln1g_ref[...], ln1b_ref[...])
    ffn = jnp.maximum(res @ fw1_ref[...] + fb1_ref[...], 0.0)
    ffn = ffn @ fw2_ref[...] + fb2_ref[...]
    out = _ln(res + ffn, ln2g_ref[...], ln2b_ref[...])

    out_ref[...] = out.reshape(BB, R, DM)
    asum_ref[...] = asum


def _stage_c(agg2, phys_p, W2_1, dev_table, Wq, Wk, Wv, Wo,
             ln1_g, ln1_b, ffn_W1, ffn_b1, ffn_W2, ffn_b2, ln2_g, ln2_b):
    full = lambda shape: pl.BlockSpec(shape, lambda i: tuple(0 for _ in shape))
    return pl.pallas_call(
        _stage_c_body,
        grid=(BP // BB,),
        in_specs=[
            pl.BlockSpec((2, BB * G, 32), lambda i: (0, i, 0)),
            pl.BlockSpec((BB, R), lambda i: (i, 0)),
            full((INTER, HF)),
            full((R, DEV)),
            full((DM, NH * DK)),
            full((DM, NH * DK)),
            full((DM, NH * DV)),
            full((NH * DV, DM)),
            full((1, DM)), full((1, DM)),
            full((DM, DI)), full((1, DI)),
            full((DI, DM)), full((1, DM)),
            full((1, DM)), full((1, DM)),
        ],
        out_specs=[
            pl.BlockSpec((BB, R, DM), lambda i: (i, 0, 0)),
            pl.BlockSpec((BB, R, R), lambda i: (i, 0, 0)),
        ],
        out_shape=[
            jax.ShapeDtypeStruct((BP, R, DM), jnp.float32),
            jax.ShapeDtypeStruct((BP, R, R), jnp.float32),
        ],
    )(agg2, phys_p, W2_1, dev_table, Wq, Wk, Wv, Wo,
      ln1_g.reshape(1, DM), ln1_b.reshape(1, DM),
      ffn_W1, ffn_b1.reshape(1, DI), ffn_W2, ffn_b2.reshape(1, DM),
      ln2_g.reshape(1, DM), ln2_b.reshape(1, DM))


def kernel(gate_types, edge_index, physical_idx, gate_embed, W1_0, W2_0,
           W1_1, W2_1, dev_table, Wq, Wk, Wv, Wo, ln1_g, ln1_b, ffn_W1,
           ffn_b1, ffn_W2, ffn_b2, ln2_g, ln2_b):
    src = edge_index[0].astype(jnp.int32)
    dst = edge_index[1].astype(jnp.int32)

    pad = E_PAD - E
    ar = jnp.arange(pad, dtype=jnp.int32)
    src_p = jnp.concatenate([src, (ar * 131) % N])
    dst_p = jnp.concatenate([dst, N + (ar % (N_SP - N))])
    src_both = jnp.stack([src_p, src_p + N]).reshape(
        2, NS * SB_TILE, SB_BLOCKS, SUB, EDGE_CHUNK)
    dst_b = dst_p.reshape(NS * SB_TILE, SB_BLOCKS, SUB, EDGE_CHUNK)

    sc_call = _make_sc_call()

    t0 = _stage_a(gate_types, gate_embed, W1_0)
    agg1 = sc_call(t0.reshape(2 * N, 32), src_both, dst_b)
    t1 = _stage_b(agg1, W2_0, W1_1)
    agg2 = sc_call(t1.reshape(2 * N, 32), src_both, dst_b)

    phys_p = jnp.concatenate(
        [physical_idx.astype(jnp.int32),
         jnp.zeros((BP - B, R), jnp.int32)], axis=0)
    outp, asum = _stage_c(agg2, phys_p, W2_1, dev_table, Wq, Wk, Wv, Wo,
                          ln1_g, ln1_b, ffn_W1, ffn_b1, ffn_W2, ffn_b2,
                          ln2_g, ln2_b)
    return outp[:B], asum[:B]


# speedup vs baseline: 10.0493x; 1.0900x over previous
import functools

import jax
import jax.numpy as jnp
from jax import lax
from jax.experimental import pallas as pl
from jax.experimental.pallas import tpu as pltpu
from jax.experimental.pallas import tpu_sc as plsc

B = 500
G = 100
Q = 32
R = 64
N = B * G
E = 800000
NUM_GATE_TYPES = 32
EMB = 64
INTER = 64
HF = 64
DEV = 64
DM = HF + DEV
NH = 4
DK = 32
DV = 32
DI = DM * 2

NC = 2
NS = 16

EDGE_CHUNK = 128
SUB = 2
BLK_E = EDGE_CHUNK * SUB
SB_BLOCKS = 10
SB_TILE = 20
E_TILE = BLK_E * SB_BLOCKS * SB_TILE
E_PAD = E_TILE * NS
N_SB = NS * SB_TILE + 1
E_IDX = N_SB * BLK_E * SB_BLOCKS
PZ = 2000
TROWS = N + PZ
ROWS_TILE = N // NS
ZROWS = 125

BN = 2000
BB = 16
BP = 512


def _stage_a_body(gt_ref, ge_ref, w1_ref, out_ref):
    tbl = jnp.maximum(ge_ref[...] @ w1_ref[...], 0.0)
    gt = gt_ref[0]
    io = lax.broadcasted_iota(jnp.int32, (NUM_GATE_TYPES, BN), 0)
    oh = (io == gt).astype(jnp.float32)
    t = lax.dot_general(oh, tbl, (((0,), (0,)), ((), ())))
    out_ref[0] = t[:, :32]
    out_ref[1] = t[:, 32:]


def _stage_a(gate_types, gate_embed, W1_0):
    gt_p = jnp.concatenate([gate_types.astype(jnp.int32),
                            jnp.full((PZ,), -1, jnp.int32)])
    gt3 = gt_p.reshape(TROWS // BN, 1, BN)
    return pl.pallas_call(
        _stage_a_body,
        grid=(TROWS // BN,),
        in_specs=[
            pl.BlockSpec((1, 1, BN), lambda i: (i, 0, 0)),
            pl.BlockSpec((NUM_GATE_TYPES, EMB), lambda i: (0, 0)),
            pl.BlockSpec((EMB, INTER), lambda i: (0, 0)),
        ],
        out_specs=pl.BlockSpec((2, BN, 32), lambda i: (0, i, 0)),
        out_shape=jax.ShapeDtypeStruct((2, TROWS, 32), jnp.float32),
    )(gt3, gate_embed, W1_0)


def _sc_body(tbl_hbm, src_hbm, dst_hbm, out_hbm,
             agg_sp, zbuf, rows0, rows1, srcv0, srcv1, dstv0, dstv1,
             sem_g0, sem_g1, sem_s0, sem_s1, sem_i0, sem_i1):
    c = lax.axis_index("c")
    s = lax.axis_index("s")

    def _z(i, carry):
        zbuf[i, pl.ds(0, 16)] = jnp.zeros((16,), jnp.float32)
        zbuf[i, pl.ds(16, 16)] = jnp.zeros((16,), jnp.float32)
        return carry
    lax.fori_loop(0, ZROWS, _z, 0)
    base_r = s * ROWS_TILE
    for k in range(ROWS_TILE // ZROWS):
        pltpu.sync_copy(zbuf, agg_sp.at[pl.ds(base_r + k * ZROWS, ZROWS)])
    plsc.subcore_barrier()

    rows = (rows0, rows1)
    srcv = (srcv0, srcv1)
    dstv = (dstv0, dstv1)
    sem_g = (sem_g0, sem_g1)
    sem_s = (sem_s0, sem_s1)
    sem_i = (sem_i0, sem_i1)

    def drain_rows(sem):
        for _ in range(SUB):
            pltpu.make_async_copy(out_hbm.at[c, pl.ds(0, EDGE_CHUNK)],
                                  rows0.at[0], sem).wait()

    def drain_idx(sem):
        pltpu.make_async_copy(src_hbm.at[c, 0], srcv0, sem).wait()
        pltpu.make_async_copy(dst_hbm.at[0], dstv0, sem).wait()

    def fire_gather(b, m, p):
        for j in range(SUB):
            pltpu.async_copy(tbl_hbm.at[srcv[b].at[m, j]], rows[p].at[j],
                             sem_g[p])

    def fire_scatter(b, m, p):
        for j in range(SUB):
            pltpu.async_copy(rows[p].at[j], agg_sp.at[dstv[b].at[m, j]],
                             sem_s[p], add=True)

    def process_sb(sbi, b, first=False):
        if not first:
            drain_idx(sem_i[b])
        for m in range(SB_BLOCKS):
            p = m % 2
            if not (first and m < 2):
                drain_rows(sem_s[p])
            fire_gather(b, m, p)
            if m == 1:
                pltpu.async_copy(src_hbm.at[c, sbi + 1], srcv[1 - b],
                                 sem_i[1 - b])
                pltpu.async_copy(dst_hbm.at[sbi + 1], dstv[1 - b],
                                 sem_i[1 - b])
            if not (first and m == 0):
                drain_rows(sem_g[1 - p])
                if m == 0:
                    fire_scatter(1 - b, SB_BLOCKS - 1, 1 - p)
                else:
                    fire_scatter(b, m - 1, 1 - p)

    base = s * SB_TILE
    pltpu.sync_copy(src_hbm.at[c, base], srcv0)
    pltpu.sync_copy(dst_hbm.at[base], dstv0)
    process_sb(base, 0, first=True)
    process_sb(base + 1, 1)

    def _pair(q, carry):
        process_sb(base + 2 * q, 0)
        process_sb(base + 2 * q + 1, 1)
        return carry
    lax.fori_loop(1, SB_TILE // 2, _pair, 0)

    drain_rows(sem_s[0])
    drain_rows(sem_g[1])
    fire_scatter(1, SB_BLOCKS - 1, 1)
    drain_rows(sem_s[1])
    drain_idx(sem_i[0])
    plsc.subcore_barrier()

    for k in range(ROWS_TILE // ZROWS):
        r0 = base_r + k * ZROWS
        pltpu.sync_copy(agg_sp.at[pl.ds(r0, ZROWS)], zbuf)
        pltpu.sync_copy(zbuf, out_hbm.at[c, pl.ds(r0, ZROWS)])


def _make_sc_call():
    mesh = plsc.VectorSubcoreMesh(core_axis_name="c", subcore_axis_name="s")
    return pl.kernel(
        _sc_body,
        out_type=jax.ShapeDtypeStruct((2, 51200, 32), jnp.float32),
        mesh=mesh,
        scratch_types=[
            pltpu.VMEM_SHARED((N, 32), jnp.float32),
            pltpu.VMEM((ZROWS, 32), jnp.float32),
            pltpu.VMEM((SUB, EDGE_CHUNK, 32), jnp.float32),
            pltpu.VMEM((SUB, EDGE_CHUNK, 32), jnp.float32),
            pltpu.VMEM((SB_BLOCKS, SUB, EDGE_CHUNK), jnp.int32),
            pltpu.VMEM((SB_BLOCKS, SUB, EDGE_CHUNK), jnp.int32),
            pltpu.VMEM((SB_BLOCKS, SUB, EDGE_CHUNK), jnp.int32),
            pltpu.VMEM((SB_BLOCKS, SUB, EDGE_CHUNK), jnp.int32),
            pltpu.SemaphoreType.DMA,
            pltpu.SemaphoreType.DMA,
            pltpu.SemaphoreType.DMA,
            pltpu.SemaphoreType.DMA,
            pltpu.SemaphoreType.DMA,
            pltpu.SemaphoreType.DMA,
        ],
        compiler_params=pltpu.CompilerParams(use_tc_tiling_on_sc=False),
    )


def _stage_b_body(agg_ref, w2_ref, w1_ref, out_ref):
    agg = jnp.concatenate([agg_ref[0], agg_ref[1]], axis=1)
    h = jnp.maximum(agg @ w2_ref[...], 0.0)
    t = jnp.maximum(h @ w1_ref[...], 0.0)
    live = jnp.where(pl.program_id(0) < N // BN, 1.0, 0.0)
    t = t * live
    out_ref[0] = t[:, :32]
    out_ref[1] = t[:, 32:]


def _stage_b(agg, W2_0, W1_1):
    nb = N // BN
    return pl.pallas_call(
        _stage_b_body,
        grid=(TROWS // BN,),
        in_specs=[
            pl.BlockSpec((2, BN, 32), lambda i: (0, jnp.minimum(i, nb - 1), 0)),
            pl.BlockSpec((INTER, HF), lambda i: (0, 0)),
            pl.BlockSpec((HF, INTER), lambda i: (0, 0)),
        ],
        out_specs=pl.BlockSpec((2, BN, 32), lambda i: (0, i, 0)),
        out_shape=jax.ShapeDtypeStruct((2, TROWS, 32), jnp.float32),
    )(agg, W2_0, W1_1)


def _stage_c_body(agg_ref, phys_ref, w2_ref, dev_ref, wq_ref, wk_ref,
                  wv_ref, wo_ref, ln1g_ref, ln1b_ref, fw1_ref, fb1_ref,
                  fw2_ref, fb2_ref, ln2g_ref, ln2b_ref, out_ref, asum_ref):
    agg = jnp.concatenate([agg_ref[0], agg_ref[1]], axis=1)
    aggq = agg.reshape(BB, G, HF)[:, :Q, :].reshape(BB * Q, HF)
    hq = jnp.maximum(aggq @ w2_ref[...], 0.0).reshape(BB, Q, HF)

    phys_q = phys_ref[...][:, :Q]
    io_r = lax.broadcasted_iota(jnp.int32, (BB, R, Q), 1)
    mask = (phys_q[:, None, :] == io_r).astype(jnp.float32)
    x_rep = lax.dot_general(mask, hq, (((2,), (1,)), ((0,), (0,))))

    dev = jnp.broadcast_to(dev_ref[...][None], (BB, R, DEV))
    x = jnp.concatenate([x_rep, dev], axis=2)
    xf = x.reshape(BB * R, DM)

    q = xf @ wq_ref[...]
    k = xf @ wk_ref[...]
    v = xf @ wv_ref[...]
    scale = 1.0 / (DK ** 0.5)

    mha = jnp.zeros((BB * R, DM), jnp.float32)
    asum = jnp.zeros((BB, R, R), jnp.float32)
    wo = wo_ref[...]
    for h in range(NH):
        qh = q[:, h * DK:(h + 1) * DK].reshape(BB, R, DK)
        kh = k[:, h * DK:(h + 1) * DK].reshape(BB, R, DK)
        vh = v[:, h * DV:(h + 1) * DV].reshape(BB, R, DV)
        s = lax.dot_general(qh, kh, (((2,), (2,)), ((0,), (0,)))) * scale
        s = s - jnp.max(s, axis=2, keepdims=True)
        es = jnp.exp(s)
        attn = es / jnp.sum(es, axis=2, keepdims=True)
        ctx = lax.dot_general(attn, vh, (((2,), (1,)), ((0,), (0,))))
        mha = mha + ctx.reshape(BB * R, DV) @ wo[h * DV:(h + 1) * DV, :]
        asum = asum + attn

    def _ln(t, g, b):
        m = jnp.mean(t, axis=1, keepdims=True)
        d = t - m
        var = jnp.mean(d * d, axis=1, keepdims=True)
        return d * lax.rsqrt(var + 1e-6) * g + b

    res = _ln(xf + mha, ln1g_ref[...], ln1b_ref[...])
    ffn = jnp.maximum(res @ fw1_ref[...] + fb1_ref[...], 0.0)
    ffn = ffn @ fw2_ref[...] + fb2_ref[...]
    out = _ln(res + ffn, ln2g_ref[...], ln2b_ref[...])

    out_ref[...] = out.reshape(BB, R, DM)
    asum_ref[...] = asum


def _stage_c(agg2, phys_p, W2_1, dev_table, Wq, Wk, Wv, Wo,
             ln1_g, ln1_b, ffn_W1, ffn_b1, ffn_W2, ffn_b2, ln2_g, ln2_b):
    full = lambda shape: pl.BlockSpec(shape, lambda i: tuple(0 for _ in shape))
    return pl.pallas_call(
        _stage_c_body,
        grid=(BP // BB,),
        in_specs=[
            pl.BlockSpec((2, BB * G, 32), lambda i: (0, i, 0)),
            pl.BlockSpec((BB, R), lambda i: (i, 0)),
            full((INTER, HF)),
            full((R, DEV)),
            full((DM, NH * DK)),
            full((DM, NH * DK)),
            full((DM, NH * DV)),
            full((NH * DV, DM)),
            full((1, DM)), full((1, DM)),
            full((DM, DI)), full((1, DI)),
            full((DI, DM)), full((1, DM)),
            full((1, DM)), full((1, DM)),
        ],
        out_specs=[
            pl.BlockSpec((BB, R, DM), lambda i: (i, 0, 0)),
            pl.BlockSpec((BB, R, R), lambda i: (i, 0, 0)),
        ],
        out_shape=[
            jax.ShapeDtypeStruct((BP, R, DM), jnp.float32),
            jax.ShapeDtypeStruct((BP, R, R), jnp.float32),
        ],
    )(agg2, phys_p, W2_1, dev_table, Wq, Wk, Wv, Wo,
      ln1_g.reshape(1, DM), ln1_b.reshape(1, DM),
      ffn_W1, ffn_b1.reshape(1, DI), ffn_W2, ffn_b2.reshape(1, DM),
      ln2_g.reshape(1, DM), ln2_b.reshape(1, DM))


def kernel(gate_types, edge_index, physical_idx, gate_embed, W1_0, W2_0,
           W1_1, W2_1, dev_table, Wq, Wk, Wv, Wo, ln1_g, ln1_b, ffn_W1,
           ffn_b1, ffn_W2, ffn_b2, ln2_g, ln2_b):
    src = edge_index[0].astype(jnp.int32)
    dst = edge_index[1].astype(jnp.int32)

    pad = E_IDX - E
    ar = jnp.arange(pad, dtype=jnp.int32)
    src_p = jnp.concatenate([src, N + (ar % PZ)])
    dst_p = jnp.concatenate([dst, (ar * 997) % N])
    src_both = jnp.stack([src_p, src_p + TROWS]).reshape(
        2, N_SB, SB_BLOCKS, SUB, EDGE_CHUNK)
    dst_b = dst_p.reshape(N_SB, SB_BLOCKS, SUB, EDGE_CHUNK)

    sc_call = _make_sc_call()

    t0 = _stage_a(gate_types, gate_embed, W1_0)
    agg1 = sc_call(t0.reshape(2 * TROWS, 32), src_both, dst_b)
    t1 = _stage_b(agg1, W2_0, W1_1)
    agg2 = sc_call(t1.reshape(2 * TROWS, 32), src_both, dst_b)

    phys_p = jnp.concatenate(
        [physical_idx.astype(jnp.int32),
         jnp.zeros((BP - B, R), jnp.int32)], axis=0)
    outp, asum = _stage_c(agg2, phys_p, W2_1, dev_table, Wq, Wk, Wv, Wo,
                          ln1_g, ln1_b, ffn_W1, ffn_b1, ffn_W2, ffn_b2,
                          ln2_g, ln2_b)
    return outp[:B], asum[:B]


# speedup vs baseline: 12.6284x; 1.2566x over previous
import numpy as np

import jax
import jax.numpy as jnp
from jax import lax
from jax.experimental import pallas as pl
from jax.experimental.pallas import tpu as pltpu
from jax.experimental.pallas import tpu_sc as plsc

B = 500
G = 100
Q = 32
R = 64
N = B * G
E = 800000
NUM_GATE_TYPES = 32
EMB = 64
INTER = 64
HF = 64
DEV = 64
DM = HF + DEV
NH = 4
DK = 32
DV = 32
DI = DM * 2

NC = 2
NS = 16

EDGE_CHUNK = 128
SUB = 2
BLK_E = EDGE_CHUNK * SUB
SB_BLOCKS = 10
SB_TILE = 20
E_TILE = BLK_E * SB_BLOCKS * SB_TILE
N_SB = NS * SB_TILE + 1
E_IDX = N_SB * BLK_E * SB_BLOCKS
PZ = 1200
TROWS = N + PZ
ROWS_TILE = N // NS
ZROWS = 125
PAD_TILE = PZ // NS

BN = 2048
BX = BN // 2
TX = TROWS // 2
BB = 16
CX = BB * G // 2

_AR = np.arange(E_IDX - E, dtype=np.int32)
_SRC_PAD = np.asarray(N + (_AR % PZ), dtype=np.int32)
_DST_PAD = np.asarray((_AR * 997) % N, dtype=np.int32)


def _bd2(w):
    z = jnp.zeros_like(w)
    return jnp.concatenate([jnp.concatenate([w, z], axis=1),
                            jnp.concatenate([z, w], axis=1)], axis=0)


def _stage_a_body(gt_ref, ge_ref, w1_ref, out_ref):
    tblf = jnp.maximum(ge_ref[...] @ w1_ref[...], 0.0)
    z = jnp.zeros_like(tblf)
    tbl2 = jnp.concatenate([jnp.concatenate([tblf, z], axis=1),
                            jnp.concatenate([z, tblf], axis=1)], axis=0)
    gtm = gt_ref[0]
    jv = lax.broadcasted_iota(jnp.int32, (BX, 2 * NUM_GATE_TYPES), 1)
    kv = jv & (NUM_GATE_TYPES - 1)
    gsel = jnp.where(jv < NUM_GATE_TYPES, gtm[:, 0:1], gtm[:, 1:2])
    oh2 = (gsel == kv).astype(jnp.float32)
    out_ref[...] = oh2 @ tbl2


def _stage_a(gate_types, gate_embed, W1_0):
    gt_p = jnp.concatenate([gate_types.astype(jnp.int32),
                            np.full((PZ,), -1, np.int32)])
    gt3 = gt_p.reshape(TROWS // BN, BX, 2)
    return pl.pallas_call(
        _stage_a_body,
        grid=(TROWS // BN,),
        in_specs=[
            pl.BlockSpec((1, BX, 2), lambda i: (i, 0, 0)),
            pl.BlockSpec((NUM_GATE_TYPES, EMB), lambda i: (0, 0)),
            pl.BlockSpec((EMB, INTER), lambda i: (0, 0)),
        ],
        out_specs=pl.BlockSpec((BX, 128), lambda i: (i, 0)),
        out_shape=jax.ShapeDtypeStruct((TX, 128), jnp.float32),
    )(gt3, gate_embed, W1_0)


def _sc_body(tbl_hbm, src_hbm, dst_hbm, out_hbm,
             agg_sp, zbuf, rows0, rows1, srcv0, srcv1, dstv0, dstv1,
             sem_g0, sem_g1, sem_s0, sem_s1, sem_i0, sem_i1):
    c = lax.axis_index("c")
    s = lax.axis_index("s")

    def _z(i, carry):
        zbuf[i, pl.ds(0, 16)] = jnp.zeros((16,), jnp.float32)
        zbuf[i, pl.ds(16, 16)] = jnp.zeros((16,), jnp.float32)
        return carry
    lax.fori_loop(0, ZROWS, _z, 0)
    base_r = s * ROWS_TILE
    for k in range(ROWS_TILE // ZROWS):
        pltpu.sync_copy(zbuf, agg_sp.at[pl.ds(base_r + k * ZROWS, ZROWS)])
    pltpu.sync_copy(zbuf.at[pl.ds(0, PAD_TILE)],
                    out_hbm.at[pl.ds(N + s * PAD_TILE, PAD_TILE), c])
    plsc.subcore_barrier()

    rows = (rows0, rows1)
    srcv = (srcv0, srcv1)
    dstv = (dstv0, dstv1)
    sem_g = (sem_g0, sem_g1)
    sem_s = (sem_s0, sem_s1)
    sem_i = (sem_i0, sem_i1)

    def drain_rows(sem):
        for _ in range(SUB):
            pltpu.make_async_copy(tbl_hbm.at[pl.ds(0, EDGE_CHUNK)],
                                  rows0.at[0], sem).wait()

    def drain_idx(sem):
        pltpu.make_async_copy(src_hbm.at[0, 0], srcv0, sem).wait()
        pltpu.make_async_copy(dst_hbm.at[0], dstv0, sem).wait()

    def fire_gather(bi, m, p):
        for j in range(SUB):
            pltpu.async_copy(tbl_hbm.at[srcv[bi].at[m, j]],
                             rows[p].at[j], sem_g[p])

    def fire_scatter(bi, m, p):
        for j in range(SUB):
            pltpu.async_copy(rows[p].at[j], agg_sp.at[dstv[bi].at[m, j]],
                             sem_s[p], add=True)

    def process_sb(sbi, bi, first=False):
        if not first:
            drain_idx(sem_i[bi])
        for m in range(SB_BLOCKS):
            p = m % 2
            if not (first and m < 2):
                drain_rows(sem_s[p])
            fire_gather(bi, m, p)
            if m == 1:
                pltpu.async_copy(src_hbm.at[c, sbi + 1], srcv[1 - bi],
                                 sem_i[1 - bi])
                pltpu.async_copy(dst_hbm.at[sbi + 1], dstv[1 - bi],
                                 sem_i[1 - bi])
            if not (first and m == 0):
                drain_rows(sem_g[1 - p])
                if m == 0:
                    fire_scatter(1 - bi, SB_BLOCKS - 1, 1 - p)
                else:
                    fire_scatter(bi, m - 1, 1 - p)

    base = s * SB_TILE
    pltpu.sync_copy(src_hbm.at[c, base], srcv0)
    pltpu.sync_copy(dst_hbm.at[base], dstv0)
    process_sb(base, 0, first=True)
    process_sb(base + 1, 1)

    def _pair(qq, carry):
        process_sb(base + 2 * qq, 0)
        process_sb(base + 2 * qq + 1, 1)
        return carry
    lax.fori_loop(1, SB_TILE // 2, _pair, 0)

    drain_rows(sem_s[0])
    drain_rows(sem_g[1])
    fire_scatter(1, SB_BLOCKS - 1, 1)
    drain_rows(sem_s[1])
    drain_idx(sem_i[0])
    plsc.subcore_barrier()

    for k in range(ROWS_TILE // ZROWS):
        r0 = base_r + k * ZROWS
        pltpu.sync_copy(agg_sp.at[pl.ds(r0, ZROWS)], zbuf)
        pltpu.sync_copy(zbuf, out_hbm.at[pl.ds(r0, ZROWS), c])


def _make_sc_call():
    mesh = plsc.VectorSubcoreMesh(core_axis_name="c", subcore_axis_name="s")
    return pl.kernel(
        _sc_body,
        out_type=jax.ShapeDtypeStruct((TROWS, 2, 32), jnp.float32),
        mesh=mesh,
        scratch_types=[
            pltpu.VMEM_SHARED((N, 32), jnp.float32),
            pltpu.VMEM((ZROWS, 32), jnp.float32),
            pltpu.VMEM((SUB, EDGE_CHUNK, 32), jnp.float32),
            pltpu.VMEM((SUB, EDGE_CHUNK, 32), jnp.float32),
            pltpu.VMEM((SB_BLOCKS, SUB, EDGE_CHUNK), jnp.int32),
            pltpu.VMEM((SB_BLOCKS, SUB, EDGE_CHUNK), jnp.int32),
            pltpu.VMEM((SB_BLOCKS, SUB, EDGE_CHUNK), jnp.int32),
            pltpu.VMEM((SB_BLOCKS, SUB, EDGE_CHUNK), jnp.int32),
            pltpu.SemaphoreType.DMA,
            pltpu.SemaphoreType.DMA,
            pltpu.SemaphoreType.DMA,
            pltpu.SemaphoreType.DMA,
            pltpu.SemaphoreType.DMA,
            pltpu.SemaphoreType.DMA,
        ],
        compiler_params=pltpu.CompilerParams(use_tc_tiling_on_sc=False),
    )


def _stage_b_body(x_in, w2_ref, w1_ref, out_ref):
    h2 = jnp.maximum(x_in[...] @ _bd2(w2_ref[...]), 0.0)
    out_ref[...] = jnp.maximum(h2 @ _bd2(w1_ref[...]), 0.0)


def _stage_b(aggx, W2_0, W1_1):
    return pl.pallas_call(
        _stage_b_body,
        grid=(TX // BX,),
        in_specs=[
            pl.BlockSpec((BX, 128), lambda i: (i, 0)),
            pl.BlockSpec((INTER, HF), lambda i: (0, 0)),
            pl.BlockSpec((HF, INTER), lambda i: (0, 0)),
        ],
        out_specs=pl.BlockSpec((BX, 128), lambda i: (i, 0)),
        out_shape=jax.ShapeDtypeStruct((TX, 128), jnp.float32),
    )(aggx, W2_0, W1_1)


def _stage_c_body(x_in, pe_ref, po_ref, w2_ref, dev_ref, wq_ref, wk_ref,
                  wv_ref, wo_ref, ln1g_ref, ln1b_ref, fw1_ref, fb1_ref,
                  fw2_ref, fb2_ref, ln2g_ref, ln2b_ref, out_ref, asum_ref):
    xq2 = x_in[...].reshape(BB, G // 2, 128)[:, :Q // 2, :]
    xq2 = xq2.reshape(BB * Q // 2, 128)
    hq2 = jnp.maximum(xq2 @ _bd2(w2_ref[...]), 0.0)

    io_r = lax.broadcasted_iota(jnp.int32, (BB, R, Q // 2), 1)
    x_rep = jnp.zeros((BB, R, HF), jnp.float32)
    for a, p_ref in ((0, pe_ref), (1, po_ref)):
        mask = (p_ref[...][:, None, :] == io_r).astype(jnp.float32)
        hq_a = hq2[:, 64 * a:64 * a + 64].reshape(BB, Q // 2, HF)
        x_rep = x_rep + lax.dot_general(
            mask, hq_a, (((2,), (1,)), ((0,), (0,))))

    dev = jnp.broadcast_to(dev_ref[...][None], (BB, R, DEV))
    x = jnp.concatenate([x_rep, dev], axis=2)
    xf = x.reshape(BB * R, DM)

    q = xf @ wq_ref[...]
    k = xf @ wk_ref[...]
    v = xf @ wv_ref[...]
    scale = 1.0 / (DK ** 0.5)

    mha = jnp.zeros((BB * R, DM), jnp.float32)
    asum = jnp.zeros((BB, R, R), jnp.float32)
    wo = wo_ref[...]
    for h in range(NH):
        qh = q[:, h * DK:(h + 1) * DK].reshape(BB, R, DK)
        kh = k[:, h * DK:(h + 1) * DK].reshape(BB, R, DK)
        vh = v[:, h * DV:(h + 1) * DV].reshape(BB, R, DV)
        s = lax.dot_general(qh, kh, (((2,), (2,)), ((0,), (0,)))) * scale
        s = s - jnp.max(s, axis=2, keepdims=True)
        es = jnp.exp(s)
        attn = es / jnp.sum(es, axis=2, keepdims=True)
        ctx = lax.dot_general(attn, vh, (((2,), (1,)), ((0,), (0,))))
        mha = mha + ctx.reshape(BB * R, DV) @ wo[h * DV:(h + 1) * DV, :]
        asum = asum + attn

    def _ln(t, g, b):
        m = jnp.mean(t, axis=1, keepdims=True)
        d = t - m
        var = jnp.mean(d * d, axis=1, keepdims=True)
        return d * lax.rsqrt(var + 1e-6) * g + b

    res = _ln(xf + mha, ln1g_ref[...], ln1b_ref[...])
    ffn = jnp.maximum(res @ fw1_ref[...] + fb1_ref[...], 0.0)
    ffn = ffn @ fw2_ref[...] + fb2_ref[...]
    out = _ln(res + ffn, ln2g_ref[...], ln2b_ref[...])

    out_ref[...] = out.reshape(BB, R, DM)
    asum_ref[...] = asum


def _stage_c(aggx, phys_e, phys_o, W2_1, dev_table, Wq, Wk, Wv, Wo,
             ln1_g, ln1_b, ffn_W1, ffn_b1, ffn_W2, ffn_b2, ln2_g, ln2_b):
    full = lambda shape: pl.BlockSpec(shape, lambda i: tuple(0 for _ in shape))
    return pl.pallas_call(
        _stage_c_body,
        grid=(TX // CX,),
        in_specs=[
            pl.BlockSpec((CX, 128), lambda i: (i, 0)),
            pl.BlockSpec((BB, Q // 2), lambda i: (i, 0)),
            pl.BlockSpec((BB, Q // 2), lambda i: (i, 0)),
            full((INTER, HF)),
            full((R, DEV)),
            full((DM, NH * DK)),
            full((DM, NH * DK)),
            full((DM, NH * DV)),
            full((NH * DV, DM)),
            full((1, DM)), full((1, DM)),
            full((DM, DI)), full((1, DI)),
            full((DI, DM)), full((1, DM)),
            full((1, DM)), full((1, DM)),
        ],
        out_specs=[
            pl.BlockSpec((BB, R, DM), lambda i: (i, 0, 0)),
            pl.BlockSpec((BB, R, R), lambda i: (i, 0, 0)),
        ],
        out_shape=[
            jax.ShapeDtypeStruct((B, R, DM), jnp.float32),
            jax.ShapeDtypeStruct((B, R, R), jnp.float32),
        ],
    )(aggx, phys_e, phys_o, W2_1, dev_table, Wq, Wk, Wv, Wo,
      ln1_g.reshape(1, DM), ln1_b.reshape(1, DM),
      ffn_W1, ffn_b1.reshape(1, DI), ffn_W2, ffn_b2.reshape(1, DM),
      ln2_g.reshape(1, DM), ln2_b.reshape(1, DM))


def kernel(gate_types, edge_index, physical_idx, gate_embed, W1_0, W2_0,
           W1_1, W2_1, dev_table, Wq, Wk, Wv, Wo, ln1_g, ln1_b, ffn_W1,
           ffn_b1, ffn_W2, ffn_b2, ln2_g, ln2_b):
    src = edge_index[0].astype(jnp.int32)
    dst = edge_index[1].astype(jnp.int32)
    src2 = 2 * jnp.concatenate([src, _SRC_PAD])
    src_b = jnp.stack([src2, src2 + 1]).reshape(
        2, N_SB, SB_BLOCKS, SUB, EDGE_CHUNK)
    dst_b = jnp.concatenate([dst, _DST_PAD]).reshape(
        N_SB, SB_BLOCKS, SUB, EDGE_CHUNK)

    sc_call = _make_sc_call()

    t0 = _stage_a(gate_types, gate_embed, W1_0)
    agg1 = sc_call(t0.reshape(2 * TROWS, 32), src_b, dst_b)
    t1 = _stage_b(agg1.reshape(TX, 128), W2_0, W1_1)
    agg2 = sc_call(t1.reshape(2 * TROWS, 32), src_b, dst_b)

    phys = physical_idx.astype(jnp.int32)
    phys_e = phys[:, 0:Q:2]
    phys_o = phys[:, 1:Q:2]
    outp, asum = _stage_c(agg2.reshape(TX, 128), phys_e, phys_o, W2_1,
                          dev_table, Wq, Wk, Wv, Wo, ln1_g, ln1_b, ffn_W1,
                          ffn_b1, ffn_W2, ffn_b2, ln2_g, ln2_b)
    return outp, asum


# speedup vs baseline: 12.8518x; 1.0177x over previous
import numpy as np

import jax
import jax.numpy as jnp
from jax import lax
from jax.experimental import pallas as pl
from jax.experimental.pallas import tpu as pltpu
from jax.experimental.pallas import tpu_sc as plsc

B = 500
G = 100
Q = 32
R = 64
N = B * G
E = 800000
NUM_GATE_TYPES = 32
EMB = 64
INTER = 64
HF = 64
DEV = 64
DM = HF + DEV
NH = 4
DK = 32
DV = 32
DI = DM * 2

NC = 2
NS = 16

EDGE_CHUNK = 128
SUB = 2
BLK_E = EDGE_CHUNK * SUB
SB_BLOCKS = 10
SB_TILE = 20
E_TILE = BLK_E * SB_BLOCKS * SB_TILE
N_SB = NS * SB_TILE + 2
IR_SB = SB_BLOCKS * SUB
E_IDX = N_SB * BLK_E * SB_BLOCKS
PZ = 1200
TROWS = N + PZ
ROWS_TILE = N // NS
ZROWS = 125
PAD_TILE = PZ // NS

BN = 2048
BX = BN // 2
TX = TROWS // 2
BB = 16
CX = BB * G // 2

_AR = np.arange(E_IDX - E, dtype=np.int32)
_SRC_PAD = np.asarray(N + (_AR % PZ), dtype=np.int32)
_DST_PAD = np.asarray((_AR * 997) % N, dtype=np.int32)


def _bd2(w):
    z = jnp.zeros_like(w)
    return jnp.concatenate([jnp.concatenate([w, z], axis=1),
                            jnp.concatenate([z, w], axis=1)], axis=0)


def _stage_a_body(gt_ref, ge_ref, w1_ref, out_ref):
    tblf = jnp.maximum(ge_ref[...] @ w1_ref[...], 0.0)
    z = jnp.zeros_like(tblf)
    tbl2 = jnp.concatenate([jnp.concatenate([tblf, z], axis=1),
                            jnp.concatenate([z, tblf], axis=1)], axis=0)
    gtm = gt_ref[0]
    jv = lax.broadcasted_iota(jnp.int32, (BX, 2 * NUM_GATE_TYPES), 1)
    kv = jv & (NUM_GATE_TYPES - 1)
    gsel = jnp.where(jv < NUM_GATE_TYPES, gtm[:, 0:1], gtm[:, 1:2])
    oh2 = (gsel == kv).astype(jnp.float32)
    out_ref[...] = oh2 @ tbl2


def _stage_a(gate_types, gate_embed, W1_0):
    gt_p = jnp.concatenate([gate_types.astype(jnp.int32),
                            np.full((PZ,), -1, np.int32)])
    gt3 = gt_p.reshape(TROWS // BN, BX, 2)
    return pl.pallas_call(
        _stage_a_body,
        grid=(TROWS // BN,),
        in_specs=[
            pl.BlockSpec((1, BX, 2), lambda i: (i, 0, 0)),
            pl.BlockSpec((NUM_GATE_TYPES, EMB), lambda i: (0, 0)),
            pl.BlockSpec((EMB, INTER), lambda i: (0, 0)),
        ],
        out_specs=pl.BlockSpec((BX, 128), lambda i: (i, 0)),
        out_shape=jax.ShapeDtypeStruct((TX, 128), jnp.float32),
    )(gt3, gate_embed, W1_0)


def _sc_body(tbl_hbm, src_hbm, dst_hbm, out_hbm,
             agg_sp, zbuf, rows0, rows1, srcv0, srcv1, dstv0, dstv1,
             sem_g0, sem_g1, sem_s0, sem_s1, sem_i0, sem_i1):
    c = lax.axis_index("c")
    s = lax.axis_index("s")

    def _z(i, carry):
        zbuf[i, pl.ds(0, 16)] = jnp.zeros((16,), jnp.float32)
        zbuf[i, pl.ds(16, 16)] = jnp.zeros((16,), jnp.float32)
        return carry
    lax.fori_loop(0, ZROWS, _z, 0)
    base_r = s * ROWS_TILE
    for k in range(ROWS_TILE // ZROWS):
        pltpu.sync_copy(zbuf, agg_sp.at[pl.ds(base_r + k * ZROWS, ZROWS)])
    pltpu.sync_copy(zbuf.at[pl.ds(0, PAD_TILE)],
                    out_hbm.at[pl.ds(N + s * PAD_TILE, PAD_TILE), c])
    plsc.subcore_barrier()

    rows = (rows0, rows1)
    srcv = (srcv0, srcv1)
    dstv = (dstv0, dstv1)
    sem_g = (sem_g0, sem_g1)
    sem_s = (sem_s0, sem_s1)
    sem_i = (sem_i0, sem_i1)

    def drain_rows(sem):
        for _ in range(SUB):
            pltpu.make_async_copy(tbl_hbm.at[pl.ds(0, EDGE_CHUNK)],
                                  rows0.at[0], sem).wait()

    def drain_idx(sem):
        pltpu.make_async_copy(src_hbm.at[0, pl.ds(0, IR_SB)], srcv0,
                              sem).wait()
        pltpu.make_async_copy(dst_hbm.at[pl.ds(0, IR_SB)], dstv0,
                              sem).wait()

    def fire_gather(bi, m, p):
        for j in range(SUB):
            pltpu.async_copy(tbl_hbm.at[srcv[bi].at[m * SUB + j]],
                             rows[p].at[j], sem_g[p])

    def fire_scatter(bi, m, p):
        for j in range(SUB):
            pltpu.async_copy(rows[p].at[j],
                             agg_sp.at[dstv[bi].at[m * SUB + j]],
                             sem_s[p], add=True)

    def process_sb(sbi, bi, first=False):
        if not first:
            drain_idx(sem_i[bi])
        for m in range(SB_BLOCKS):
            p = m % 2
            if not (first and m < 2):
                drain_rows(sem_s[p])
            fire_gather(bi, m, p)
            if m == 1:
                pltpu.async_copy(
                    src_hbm.at[c, pl.ds((sbi + 1) * IR_SB, IR_SB)],
                    srcv[1 - bi], sem_i[1 - bi])
                pltpu.async_copy(
                    dst_hbm.at[pl.ds((sbi + 1) * IR_SB, IR_SB)],
                    dstv[1 - bi], sem_i[1 - bi])
            if not (first and m == 0):
                drain_rows(sem_g[1 - p])
                if m == 0:
                    fire_scatter(1 - bi, SB_BLOCKS - 1, 1 - p)
                else:
                    fire_scatter(bi, m - 1, 1 - p)

    base = s * SB_TILE
    pltpu.sync_copy(src_hbm.at[c, pl.ds(base * IR_SB, IR_SB)], srcv0)
    pltpu.sync_copy(dst_hbm.at[pl.ds(base * IR_SB, IR_SB)], dstv0)
    process_sb(base, 0, first=True)
    process_sb(base + 1, 1)

    def _pair(qq, carry):
        process_sb(base + 2 * qq, 0)
        process_sb(base + 2 * qq + 1, 1)
        return carry
    lax.fori_loop(1, SB_TILE // 2, _pair, 0)

    drain_rows(sem_s[0])
    drain_rows(sem_g[1])
    fire_scatter(1, SB_BLOCKS - 1, 1)
    drain_rows(sem_s[1])
    drain_idx(sem_i[0])
    plsc.subcore_barrier()

    for k in range(ROWS_TILE // ZROWS):
        r0 = base_r + k * ZROWS
        pltpu.sync_copy(agg_sp.at[pl.ds(r0, ZROWS)], zbuf)
        pltpu.sync_copy(zbuf, out_hbm.at[pl.ds(r0, ZROWS), c])


def _make_sc_call():
    mesh = plsc.VectorSubcoreMesh(core_axis_name="c", subcore_axis_name="s")
    return pl.kernel(
        _sc_body,
        out_type=jax.ShapeDtypeStruct((TROWS, 2, 32), jnp.float32),
        mesh=mesh,
        scratch_types=[
            pltpu.VMEM_SHARED((N, 32), jnp.float32),
            pltpu.VMEM((ZROWS, 32), jnp.float32),
            pltpu.VMEM((SUB, EDGE_CHUNK, 32), jnp.float32),
            pltpu.VMEM((SUB, EDGE_CHUNK, 32), jnp.float32),
            pltpu.VMEM((IR_SB, EDGE_CHUNK), jnp.int32),
            pltpu.VMEM((IR_SB, EDGE_CHUNK), jnp.int32),
            pltpu.VMEM((IR_SB, EDGE_CHUNK), jnp.int32),
            pltpu.VMEM((IR_SB, EDGE_CHUNK), jnp.int32),
            pltpu.SemaphoreType.DMA,
            pltpu.SemaphoreType.DMA,
            pltpu.SemaphoreType.DMA,
            pltpu.SemaphoreType.DMA,
            pltpu.SemaphoreType.DMA,
            pltpu.SemaphoreType.DMA,
        ],
        compiler_params=pltpu.CompilerParams(use_tc_tiling_on_sc=False),
    )


def _stage_b_body(x_in, w2_ref, w1_ref, out_ref):
    h2 = jnp.maximum(x_in[...] @ _bd2(w2_ref[...]), 0.0)
    out_ref[...] = jnp.maximum(h2 @ _bd2(w1_ref[...]), 0.0)


def _stage_b(aggx, W2_0, W1_1):
    return pl.pallas_call(
        _stage_b_body,
        grid=(TX // BX,),
        in_specs=[
            pl.BlockSpec((BX, 128), lambda i: (i, 0)),
            pl.BlockSpec((INTER, HF), lambda i: (0, 0)),
            pl.BlockSpec((HF, INTER), lambda i: (0, 0)),
        ],
        out_specs=pl.BlockSpec((BX, 128), lambda i: (i, 0)),
        out_shape=jax.ShapeDtypeStruct((TX, 128), jnp.float32),
    )(aggx, W2_0, W1_1)


def _stage_c_body(x_in, pe_ref, po_ref, w2_ref, dev_ref, wq_ref, wk_ref,
                  wv_ref, wo_ref, ln1g_ref, ln1b_ref, fw1_ref, fb1_ref,
                  fw2_ref, fb2_ref, ln2g_ref, ln2b_ref, out_ref, asum_ref):
    xq2 = x_in[...].reshape(BB, G // 2, 128)[:, :Q // 2, :]
    xq2 = xq2.reshape(BB * Q // 2, 128)
    hq2 = jnp.maximum(xq2 @ _bd2(w2_ref[...]), 0.0)

    io_r = lax.broadcasted_iota(jnp.int32, (BB, R, Q // 2), 1)
    x_rep = jnp.zeros((BB, R, HF), jnp.float32)
    for a, p_ref in ((0, pe_ref), (1, po_ref)):
        mask = (p_ref[...][:, None, :] == io_r).astype(jnp.float32)
        hq_a = hq2[:, 64 * a:64 * a + 64].reshape(BB, Q // 2, HF)
        x_rep = x_rep + lax.dot_general(
            mask, hq_a, (((2,), (1,)), ((0,), (0,))))

    dev = jnp.broadcast_to(dev_ref[...][None], (BB, R, DEV))
    x = jnp.concatenate([x_rep, dev], axis=2)
    xf = x.reshape(BB * R, DM)

    q = xf @ wq_ref[...]
    k = xf @ wk_ref[...]
    v = xf @ wv_ref[...]
    scale = 1.0 / (DK ** 0.5)

    mha = jnp.zeros((BB * R, DM), jnp.float32)
    asum = jnp.zeros((BB, R, R), jnp.float32)
    wo = wo_ref[...]
    for h in range(NH):
        qh = q[:, h * DK:(h + 1) * DK].reshape(BB, R, DK)
        kh = k[:, h * DK:(h + 1) * DK].reshape(BB, R, DK)
        vh = v[:, h * DV:(h + 1) * DV].reshape(BB, R, DV)
        s = lax.dot_general(qh, kh, (((2,), (2,)), ((0,), (0,)))) * scale
        s = s - jnp.max(s, axis=2, keepdims=True)
        es = jnp.exp(s)
        attn = es / jnp.sum(es, axis=2, keepdims=True)
        ctx = lax.dot_general(attn, vh, (((2,), (1,)), ((0,), (0,))))
        mha = mha + ctx.reshape(BB * R, DV) @ wo[h * DV:(h + 1) * DV, :]
        asum = asum + attn

    def _ln(t, g, b):
        m = jnp.mean(t, axis=1, keepdims=True)
        d = t - m
        var = jnp.mean(d * d, axis=1, keepdims=True)
        return d * lax.rsqrt(var + 1e-6) * g + b

    res = _ln(xf + mha, ln1g_ref[...], ln1b_ref[...])
    ffn = jnp.maximum(res @ fw1_ref[...] + fb1_ref[...], 0.0)
    ffn = ffn @ fw2_ref[...] + fb2_ref[...]
    out = _ln(res + ffn, ln2g_ref[...], ln2b_ref[...])

    out_ref[...] = out.reshape(BB, R, DM)
    asum_ref[...] = asum


def _stage_c(aggx, phys_e, phys_o, W2_1, dev_table, Wq, Wk, Wv, Wo,
             ln1_g, ln1_b, ffn_W1, ffn_b1, ffn_W2, ffn_b2, ln2_g, ln2_b):
    full = lambda shape: pl.BlockSpec(shape, lambda i: tuple(0 for _ in shape))
    return pl.pallas_call(
        _stage_c_body,
        grid=(TX // CX,),
        in_specs=[
            pl.BlockSpec((CX, 128), lambda i: (i, 0)),
            pl.BlockSpec((BB, Q // 2), lambda i: (i, 0)),
            pl.BlockSpec((BB, Q // 2), lambda i: (i, 0)),
            full((INTER, HF)),
            full((R, DEV)),
            full((DM, NH * DK)),
            full((DM, NH * DK)),
            full((DM, NH * DV)),
            full((NH * DV, DM)),
            full((1, DM)), full((1, DM)),
            full((DM, DI)), full((1, DI)),
            full((DI, DM)), full((1, DM)),
            full((1, DM)), full((1, DM)),
        ],
        out_specs=[
            pl.BlockSpec((BB, R, DM), lambda i: (i, 0, 0)),
            pl.BlockSpec((BB, R, R), lambda i: (i, 0, 0)),
        ],
        out_shape=[
            jax.ShapeDtypeStruct((B, R, DM), jnp.float32),
            jax.ShapeDtypeStruct((B, R, R), jnp.float32),
        ],
    )(aggx, phys_e, phys_o, W2_1, dev_table, Wq, Wk, Wv, Wo,
      ln1_g.reshape(1, DM), ln1_b.reshape(1, DM),
      ffn_W1, ffn_b1.reshape(1, DI), ffn_W2, ffn_b2.reshape(1, DM),
      ln2_g.reshape(1, DM), ln2_b.reshape(1, DM))


def kernel(gate_types, edge_index, physical_idx, gate_embed, W1_0, W2_0,
           W1_1, W2_1, dev_table, Wq, Wk, Wv, Wo, ln1_g, ln1_b, ffn_W1,
           ffn_b1, ffn_W2, ffn_b2, ln2_g, ln2_b):
    src = edge_index[0].astype(jnp.int32)
    dst = edge_index[1].astype(jnp.int32)
    src2 = 2 * jnp.concatenate([src, _SRC_PAD])
    src_b = jnp.stack([src2, src2 + 1]).reshape(
        2, N_SB * IR_SB, EDGE_CHUNK)
    dst_b = jnp.concatenate([dst, _DST_PAD]).reshape(
        N_SB * IR_SB, EDGE_CHUNK)

    sc_call = _make_sc_call()

    t0 = _stage_a(gate_types, gate_embed, W1_0)
    agg1 = sc_call(t0.reshape(2 * TROWS, 32), src_b, dst_b)
    t1 = _stage_b(agg1.reshape(TX, 128), W2_0, W1_1)
    agg2 = sc_call(t1.reshape(2 * TROWS, 32), src_b, dst_b)

    phys = physical_idx.astype(jnp.int32)
    phys_e = phys[:, 0:Q:2]
    phys_o = phys[:, 1:Q:2]
    outp, asum = _stage_c(agg2.reshape(TX, 128), phys_e, phys_o, W2_1,
                          dev_table, Wq, Wk, Wv, Wo, ln1_g, ln1_b, ffn_W1,
                          ffn_b1, ffn_W2, ffn_b2, ln2_g, ln2_b)
    return outp, asum


# speedup vs baseline: 13.8818x; 1.0801x over previous
import numpy as np

import jax
import jax.numpy as jnp
from jax import lax
from jax.experimental import pallas as pl
from jax.experimental.pallas import tpu as pltpu
from jax.experimental.pallas import tpu_sc as plsc

B = 500
G = 100
Q = 32
R = 64
N = B * G
E = 800000
NUM_GATE_TYPES = 32
EMB = 64
INTER = 64
HF = 64
DEV = 64
DM = HF + DEV
NH = 4
DK = 32
DV = 32
DI = DM * 2

NC = 2
NS = 16

EDGE_CHUNK = 128
SUB = 2
BLK_E = EDGE_CHUNK * SUB
SB_BLOCKS = 10
SB_TILE = 20
E_TILE = BLK_E * SB_BLOCKS * SB_TILE
N_SB = NS * SB_TILE + 2
IR_SB = SB_BLOCKS * SUB
E_IDX = N_SB * BLK_E * SB_BLOCKS
PZ = 1200
TROWS = N + PZ
ROWS_TILE = N // NS
ZROWS = 125
PAD_TILE = PZ // NS

BN = 2048
BX = BN // 2
TX = TROWS // 2
BB = 32
CX = BB * G // 2

_AR = np.arange(E_IDX - E, dtype=np.int32)
_SRC_PAD = np.asarray(N + (_AR % PZ), dtype=np.int32)
_DST_PAD = np.asarray((_AR * 997) % N, dtype=np.int32)


def _bd2(w):
    z = jnp.zeros_like(w)
    return jnp.concatenate([jnp.concatenate([w, z], axis=1),
                            jnp.concatenate([z, w], axis=1)], axis=0)


def _stage_a_body(gt_ref, ge_ref, w1_ref, out_ref):
    tblf = jnp.maximum(ge_ref[...] @ w1_ref[...], 0.0)
    z = jnp.zeros_like(tblf)
    tbl2 = jnp.concatenate([jnp.concatenate([tblf, z], axis=1),
                            jnp.concatenate([z, tblf], axis=1)], axis=0)
    gtm = gt_ref[0]
    jv = lax.broadcasted_iota(jnp.int32, (BX, 2 * NUM_GATE_TYPES), 1)
    kv = jv & (NUM_GATE_TYPES - 1)
    gsel = jnp.where(jv < NUM_GATE_TYPES, gtm[:, 0:1], gtm[:, 1:2])
    oh2 = (gsel == kv).astype(jnp.float32)
    out_ref[...] = oh2 @ tbl2


def _stage_a(gate_types, gate_embed, W1_0):
    gt_p = jnp.concatenate([gate_types.astype(jnp.int32),
                            np.full((PZ,), -1, np.int32)])
    gt3 = gt_p.reshape(TROWS // BN, BX, 2)
    return pl.pallas_call(
        _stage_a_body,
        grid=(TROWS // BN,),
        in_specs=[
            pl.BlockSpec((1, BX, 2), lambda i: (i, 0, 0)),
            pl.BlockSpec((NUM_GATE_TYPES, EMB), lambda i: (0, 0)),
            pl.BlockSpec((EMB, INTER), lambda i: (0, 0)),
        ],
        out_specs=pl.BlockSpec((BX, 128), lambda i: (i, 0)),
        out_shape=jax.ShapeDtypeStruct((TX, 128), jnp.float32),
    )(gt3, gate_embed, W1_0)


def _sc_body(tbl_hbm, src_hbm, dst_hbm, out_hbm,
             agg_sp, zbuf, rows0, rows1, srcv0, srcv1, dstv0, dstv1,
             sem_g0, sem_g1, sem_s0, sem_s1, sem_i0, sem_i1):
    c = lax.axis_index("c")
    s = lax.axis_index("s")

    def _z(i, carry):
        zbuf[i, pl.ds(0, 16)] = jnp.zeros((16,), jnp.float32)
        zbuf[i, pl.ds(16, 16)] = jnp.zeros((16,), jnp.float32)
        return carry
    lax.fori_loop(0, ZROWS, _z, 0)
    base_r = s * ROWS_TILE
    for k in range(ROWS_TILE // ZROWS):
        pltpu.sync_copy(zbuf, agg_sp.at[pl.ds(base_r + k * ZROWS, ZROWS)])
    pltpu.sync_copy(zbuf.at[pl.ds(0, PAD_TILE)],
                    out_hbm.at[pl.ds(N + s * PAD_TILE, PAD_TILE), c])
    plsc.subcore_barrier()

    rows = (rows0, rows1)
    srcv = (srcv0, srcv1)
    dstv = (dstv0, dstv1)
    sem_g = (sem_g0, sem_g1)
    sem_s = (sem_s0, sem_s1)
    sem_i = (sem_i0, sem_i1)

    def drain_rows(sem):
        for _ in range(SUB):
            pltpu.make_async_copy(tbl_hbm.at[pl.ds(0, EDGE_CHUNK)],
                                  rows0.at[0], sem).wait()

    def drain_idx(sem):
        pltpu.make_async_copy(src_hbm.at[0, pl.ds(0, IR_SB)], srcv0,
                              sem).wait()
        pltpu.make_async_copy(dst_hbm.at[pl.ds(0, IR_SB)], dstv0,
                              sem).wait()

    def fire_gather(bi, m, p):
        for j in range(SUB):
            pltpu.async_copy(tbl_hbm.at[srcv[bi].at[m * SUB + j]],
                             rows[p].at[j], sem_g[p])

    def fire_scatter(bi, m, p):
        for j in range(SUB):
            pltpu.async_copy(rows[p].at[j],
                             agg_sp.at[dstv[bi].at[m * SUB + j]],
                             sem_s[p], add=True)

    def process_sb(sbi, bi, first=False):
        if not first:
            drain_idx(sem_i[bi])
        for m in range(SB_BLOCKS):
            p = m % 2
            if not (first and m < 2):
                drain_rows(sem_s[p])
            fire_gather(bi, m, p)
            if m == 1:
                pltpu.async_copy(
                    src_hbm.at[c, pl.ds((sbi + 1) * IR_SB, IR_SB)],
                    srcv[1 - bi], sem_i[1 - bi])
                pltpu.async_copy(
                    dst_hbm.at[pl.ds((sbi + 1) * IR_SB, IR_SB)],
                    dstv[1 - bi], sem_i[1 - bi])
            if not (first and m == 0):
                drain_rows(sem_g[1 - p])
                if m == 0:
                    fire_scatter(1 - bi, SB_BLOCKS - 1, 1 - p)
                else:
                    fire_scatter(bi, m - 1, 1 - p)

    base = s * SB_TILE
    pltpu.sync_copy(src_hbm.at[c, pl.ds(base * IR_SB, IR_SB)], srcv0)
    pltpu.sync_copy(dst_hbm.at[pl.ds(base * IR_SB, IR_SB)], dstv0)
    process_sb(base, 0, first=True)
    process_sb(base + 1, 1)

    def _pair(qq, carry):
        process_sb(base + 2 * qq, 0)
        process_sb(base + 2 * qq + 1, 1)
        return carry
    lax.fori_loop(1, SB_TILE // 2, _pair, 0)

    drain_rows(sem_s[0])
    drain_rows(sem_g[1])
    fire_scatter(1, SB_BLOCKS - 1, 1)
    drain_rows(sem_s[1])
    drain_idx(sem_i[0])
    plsc.subcore_barrier()

    for k in range(ROWS_TILE // ZROWS):
        r0 = base_r + k * ZROWS
        pltpu.sync_copy(agg_sp.at[pl.ds(r0, ZROWS)], zbuf)
        pltpu.sync_copy(zbuf, out_hbm.at[pl.ds(r0, ZROWS), c])


def _make_sc_call():
    mesh = plsc.VectorSubcoreMesh(core_axis_name="c", subcore_axis_name="s")
    return pl.kernel(
        _sc_body,
        out_type=jax.ShapeDtypeStruct((TROWS, 2, 32), jnp.float32),
        mesh=mesh,
        scratch_types=[
            pltpu.VMEM_SHARED((N, 32), jnp.float32),
            pltpu.VMEM((ZROWS, 32), jnp.float32),
            pltpu.VMEM((SUB, EDGE_CHUNK, 32), jnp.float32),
            pltpu.VMEM((SUB, EDGE_CHUNK, 32), jnp.float32),
            pltpu.VMEM((IR_SB, EDGE_CHUNK), jnp.int32),
            pltpu.VMEM((IR_SB, EDGE_CHUNK), jnp.int32),
            pltpu.VMEM((IR_SB, EDGE_CHUNK), jnp.int32),
            pltpu.VMEM((IR_SB, EDGE_CHUNK), jnp.int32),
            pltpu.SemaphoreType.DMA,
            pltpu.SemaphoreType.DMA,
            pltpu.SemaphoreType.DMA,
            pltpu.SemaphoreType.DMA,
            pltpu.SemaphoreType.DMA,
            pltpu.SemaphoreType.DMA,
        ],
        compiler_params=pltpu.CompilerParams(use_tc_tiling_on_sc=False),
    )


def _stage_b_body(x_in, w2_ref, w1_ref, out_ref):
    h2 = jnp.maximum(x_in[...] @ _bd2(w2_ref[...]), 0.0)
    out_ref[...] = jnp.maximum(h2 @ _bd2(w1_ref[...]), 0.0)


def _stage_b(aggx, W2_0, W1_1):
    return pl.pallas_call(
        _stage_b_body,
        grid=(TX // BX,),
        in_specs=[
            pl.BlockSpec((BX, 128), lambda i: (i, 0)),
            pl.BlockSpec((INTER, HF), lambda i: (0, 0)),
            pl.BlockSpec((HF, INTER), lambda i: (0, 0)),
        ],
        out_specs=pl.BlockSpec((BX, 128), lambda i: (i, 0)),
        out_shape=jax.ShapeDtypeStruct((TX, 128), jnp.float32),
    )(aggx, W2_0, W1_1)


def _stage_c_body(x_in, pe_ref, po_ref, w2_ref, dev_ref, wq_ref, wk_ref,
                  wv_ref, wo_ref, ln1g_ref, ln1b_ref, fw1_ref, fb1_ref,
                  fw2_ref, fb2_ref, ln2g_ref, ln2b_ref, out_ref, asum_ref):
    xq2 = x_in[...].reshape(BB, G // 2, 128)[:, :Q // 2, :]
    xq2 = xq2.reshape(BB * Q // 2, 128)
    hq2 = jnp.maximum(xq2 @ _bd2(w2_ref[...]), 0.0)

    io_r = lax.broadcasted_iota(jnp.int32, (BB, R, Q // 2), 1)
    x_rep = jnp.zeros((BB, R, HF), jnp.float32)
    for a, p_ref in ((0, pe_ref), (1, po_ref)):
        mask = (p_ref[...][:, None, :] == io_r).astype(jnp.float32)
        hq_a = hq2[:, 64 * a:64 * a + 64].reshape(BB, Q // 2, HF)
        x_rep = x_rep + lax.dot_general(
            mask, hq_a, (((2,), (1,)), ((0,), (0,))))

    dev = jnp.broadcast_to(dev_ref[...][None], (BB, R, DEV))
    x = jnp.concatenate([x_rep, dev], axis=2)
    xf = x.reshape(BB * R, DM)

    q = (xf @ wq_ref[...]) * (1.0 / (DK ** 0.5))
    k = xf @ wk_ref[...]
    v = xf @ wv_ref[...]

    ctxs = []
    asum = jnp.zeros((BB, R, R), jnp.float32)
    for h in range(NH):
        qh = q[:, h * DK:(h + 1) * DK].reshape(BB, R, DK)
        kh = k[:, h * DK:(h + 1) * DK].reshape(BB, R, DK)
        vh = v[:, h * DV:(h + 1) * DV].reshape(BB, R, DV)
        s = lax.dot_general(qh, kh, (((2,), (2,)), ((0,), (0,))))
        s = s - jnp.max(s, axis=2, keepdims=True)
        es = jnp.exp(s)
        attn = es / jnp.sum(es, axis=2, keepdims=True)
        ctx = lax.dot_general(attn, vh, (((2,), (1,)), ((0,), (0,))))
        ctxs.append(ctx.reshape(BB * R, DV))
        asum = asum + attn
    mha = jnp.concatenate(ctxs, axis=1) @ wo_ref[...]

    def _ln(t, g, b):
        m = jnp.mean(t, axis=1, keepdims=True)
        d = t - m
        var = jnp.mean(d * d, axis=1, keepdims=True)
        return d * lax.rsqrt(var + 1e-6) * g + b

    res = _ln(xf + mha, ln1g_ref[...], ln1b_ref[...])
    ffn = jnp.maximum(res @ fw1_ref[...] + fb1_ref[...], 0.0)
    ffn = ffn @ fw2_ref[...] + fb2_ref[...]
    out = _ln(res + ffn, ln2g_ref[...], ln2b_ref[...])

    out_ref[...] = out.reshape(BB, R, DM)
    asum_ref[...] = asum


def _stage_c(aggx, phys_e, phys_o, W2_1, dev_table, Wq, Wk, Wv, Wo,
             ln1_g, ln1_b, ffn_W1, ffn_b1, ffn_W2, ffn_b2, ln2_g, ln2_b):
    full = lambda shape: pl.BlockSpec(shape, lambda i: tuple(0 for _ in shape))
    return pl.pallas_call(
        _stage_c_body,
        grid=(TX // CX,),
        in_specs=[
            pl.BlockSpec((CX, 128), lambda i: (i, 0)),
            pl.BlockSpec((BB, Q // 2), lambda i: (i, 0)),
            pl.BlockSpec((BB, Q // 2), lambda i: (i, 0)),
            full((INTER, HF)),
            full((R, DEV)),
            full((DM, NH * DK)),
            full((DM, NH * DK)),
            full((DM, NH * DV)),
            full((NH * DV, DM)),
            full((1, DM)), full((1, DM)),
            full((DM, DI)), full((1, DI)),
            full((DI, DM)), full((1, DM)),
            full((1, DM)), full((1, DM)),
        ],
        out_specs=[
            pl.BlockSpec((BB, R, DM), lambda i: (i, 0, 0)),
            pl.BlockSpec((BB, R, R), lambda i: (i, 0, 0)),
        ],
        out_shape=[
            jax.ShapeDtypeStruct((B, R, DM), jnp.float32),
            jax.ShapeDtypeStruct((B, R, R), jnp.float32),
        ],
    )(aggx, phys_e, phys_o, W2_1, dev_table, Wq, Wk, Wv, Wo,
      ln1_g.reshape(1, DM), ln1_b.reshape(1, DM),
      ffn_W1, ffn_b1.reshape(1, DI), ffn_W2, ffn_b2.reshape(1, DM),
      ln2_g.reshape(1, DM), ln2_b.reshape(1, DM))


def kernel(gate_types, edge_index, physical_idx, gate_embed, W1_0, W2_0,
           W1_1, W2_1, dev_table, Wq, Wk, Wv, Wo, ln1_g, ln1_b, ffn_W1,
           ffn_b1, ffn_W2, ffn_b2, ln2_g, ln2_b):
    src = edge_index[0].astype(jnp.int32)
    dst = edge_index[1].astype(jnp.int32)
    src2 = 2 * jnp.concatenate([src, _SRC_PAD])
    src_b = jnp.stack([src2, src2 + 1]).reshape(
        2, N_SB * IR_SB, EDGE_CHUNK)
    dst_b = jnp.concatenate([dst, _DST_PAD]).reshape(
        N_SB * IR_SB, EDGE_CHUNK)

    sc_call = _make_sc_call()

    t0 = _stage_a(gate_types, gate_embed, W1_0)
    agg1 = sc_call(t0.reshape(2 * TROWS, 32), src_b, dst_b)
    t1 = _stage_b(agg1.reshape(TX, 128), W2_0, W1_1)
    agg2 = sc_call(t1.reshape(2 * TROWS, 32), src_b, dst_b)

    phys = physical_idx.astype(jnp.int32)
    phys_e = phys[:, 0:Q:2]
    phys_o = phys[:, 1:Q:2]
    outp, asum = _stage_c(agg2.reshape(TX, 128), phys_e, phys_o, W2_1,
                          dev_table, Wq, Wk, Wv, Wo, ln1_g, ln1_b, ffn_W1,
                          ffn_b1, ffn_W2, ffn_b2, ln2_g, ln2_b)
    return outp, asum


# speedup vs baseline: 14.7888x; 1.0653x over previous
import numpy as np

import jax
import jax.numpy as jnp
from jax import lax
from jax.experimental import pallas as pl
from jax.experimental.pallas import tpu as pltpu
from jax.experimental.pallas import tpu_sc as plsc

B = 500
G = 100
Q = 32
R = 64
N = B * G
E = 800000
NUM_GATE_TYPES = 32
EMB = 64
INTER = 64
HF = 64
DEV = 64
DM = HF + DEV
NH = 4
DK = 32
DV = 32
DI = DM * 2

NC = 2
NS = 16

EDGE_CHUNK = 128
SUB = 2
BLK_E = EDGE_CHUNK * SUB
SB_BLOCKS = 10
SB_TILE = 20
E_TILE = BLK_E * SB_BLOCKS * SB_TILE
N_SB = NS * SB_TILE + 2
IR_SB = SB_BLOCKS * SUB
E_IDX = N_SB * BLK_E * SB_BLOCKS
PZ = 1200
TROWS = N + PZ
ROWS_TILE = N // NS
ZROWS = 125
PAD_TILE = PZ // NS

BN = 2048
BX = BN // 2
TX = TROWS // 2
BB = 32
CX = BB * G // 2

PR = 280
PE = PR * EDGE_CHUNK

_AR = np.arange(E_IDX - E, dtype=np.int32)
_SRC_PAD = np.asarray(N + (_AR % PZ), dtype=np.int32)
_DST_PAD = np.asarray((_AR * 997) % N, dtype=np.int32)


def _prep_body(ei_ref, src_ref, dst_ref):
    g = pl.program_id(0)
    s2d = ei_ref[0].reshape(PR, EDGE_CHUNK)
    d2d = ei_ref[1].reshape(PR, EDGE_CHUNK)
    fr = lax.broadcasted_iota(jnp.int32, (PR, EDGE_CHUNK), 0)
    fl = lax.broadcasted_iota(jnp.int32, (PR, EDGE_CHUNK), 1)
    flat = g * PE + fr * EDGE_CHUNK + fl
    real = flat < E
    pad = flat - E
    sv = jnp.where(real, s2d, N + lax.rem(pad, PZ))
    dv = jnp.where(real, d2d, lax.rem(pad * 997, N))
    src_ref[0] = 2 * sv
    src_ref[1] = 2 * sv + 1
    dst_ref[...] = dv


def _prep(edge_index):
    return pl.pallas_call(
        _prep_body,
        grid=(N_SB * IR_SB // PR,),
        in_specs=[pl.BlockSpec((2, PE), lambda i: (0, i))],
        out_specs=[
            pl.BlockSpec((2, PR, EDGE_CHUNK), lambda i: (0, i, 0)),
            pl.BlockSpec((PR, EDGE_CHUNK), lambda i: (i, 0)),
        ],
        out_shape=[
            jax.ShapeDtypeStruct((2, N_SB * IR_SB, EDGE_CHUNK), jnp.int32),
            jax.ShapeDtypeStruct((N_SB * IR_SB, EDGE_CHUNK), jnp.int32),
        ],
    )(edge_index.astype(jnp.int32))


def _bd2(w):
    z = jnp.zeros_like(w)
    return jnp.concatenate([jnp.concatenate([w, z], axis=1),
                            jnp.concatenate([z, w], axis=1)], axis=0)


def _stage_a_body(gt_ref, ge_ref, w1_ref, out_ref):
    tblf = jnp.maximum(ge_ref[...] @ w1_ref[...], 0.0)
    z = jnp.zeros_like(tblf)
    tbl2 = jnp.concatenate([jnp.concatenate([tblf, z], axis=1),
                            jnp.concatenate([z, tblf], axis=1)], axis=0)
    gtm = gt_ref[0]
    jv = lax.broadcasted_iota(jnp.int32, (BX, 2 * NUM_GATE_TYPES), 1)
    kv = jv & (NUM_GATE_TYPES - 1)
    gsel = jnp.where(jv < NUM_GATE_TYPES, gtm[:, 0:1], gtm[:, 1:2])
    oh2 = (gsel == kv).astype(jnp.float32)
    out_ref[...] = oh2 @ tbl2


def _stage_a(gate_types, gate_embed, W1_0):
    gt_p = jnp.concatenate([gate_types.astype(jnp.int32),
                            np.full((PZ,), -1, np.int32)])
    gt3 = gt_p.reshape(TROWS // BN, BX, 2)
    return pl.pallas_call(
        _stage_a_body,
        grid=(TROWS // BN,),
        in_specs=[
            pl.BlockSpec((1, BX, 2), lambda i: (i, 0, 0)),
            pl.BlockSpec((NUM_GATE_TYPES, EMB), lambda i: (0, 0)),
            pl.BlockSpec((EMB, INTER), lambda i: (0, 0)),
        ],
        out_specs=pl.BlockSpec((BX, 128), lambda i: (i, 0)),
        out_shape=jax.ShapeDtypeStruct((TX, 128), jnp.float32),
    )(gt3, gate_embed, W1_0)


def _sc_body(tbl_hbm, src_hbm, dst_hbm, out_hbm,
             agg_sp, zbuf, rows0, rows1, srcv0, srcv1, dstv0, dstv1,
             sem_g0, sem_g1, sem_s0, sem_s1, sem_i0, sem_i1):
    c = lax.axis_index("c")
    s = lax.axis_index("s")

    def _z(i, carry):
        zbuf[i, pl.ds(0, 16)] = jnp.zeros((16,), jnp.float32)
        zbuf[i, pl.ds(16, 16)] = jnp.zeros((16,), jnp.float32)
        return carry
    lax.fori_loop(0, ZROWS, _z, 0)
    base_r = s * ROWS_TILE
    for k in range(ROWS_TILE // ZROWS):
        pltpu.sync_copy(zbuf, agg_sp.at[pl.ds(base_r + k * ZROWS, ZROWS)])
    pltpu.sync_copy(zbuf.at[pl.ds(0, PAD_TILE)],
                    out_hbm.at[pl.ds(N + s * PAD_TILE, PAD_TILE), c])
    plsc.subcore_barrier()

    rows = (rows0, rows1)
    srcv = (srcv0, srcv1)
    dstv = (dstv0, dstv1)
    sem_g = (sem_g0, sem_g1)
    sem_s = (sem_s0, sem_s1)
    sem_i = (sem_i0, sem_i1)

    def drain_rows(sem):
        for _ in range(SUB):
            pltpu.make_async_copy(tbl_hbm.at[pl.ds(0, EDGE_CHUNK)],
                                  rows0.at[0], sem).wait()

    def drain_idx(sem):
        pltpu.make_async_copy(src_hbm.at[0, pl.ds(0, IR_SB)], srcv0,
                              sem).wait()
        pltpu.make_async_copy(dst_hbm.at[pl.ds(0, IR_SB)], dstv0,
                              sem).wait()

    def fire_gather(bi, m, p):
        for j in range(SUB):
            pltpu.async_copy(tbl_hbm.at[srcv[bi].at[m * SUB + j]],
                             rows[p].at[j], sem_g[p])

    def fire_scatter(bi, m, p):
        for j in range(SUB):
            pltpu.async_copy(rows[p].at[j],
                             agg_sp.at[dstv[bi].at[m * SUB + j]],
                             sem_s[p], add=True)

    def process_sb(sbi, bi, first=False):
        if not first:
            drain_idx(sem_i[bi])
        for m in range(SB_BLOCKS):
            p = m % 2
            if not (first and m < 2):
                drain_rows(sem_s[p])
            fire_gather(bi, m, p)
            if m == 1:
                pltpu.async_copy(
                    src_hbm.at[c, pl.ds((sbi + 1) * IR_SB, IR_SB)],
                    srcv[1 - bi], sem_i[1 - bi])
                pltpu.async_copy(
                    dst_hbm.at[pl.ds((sbi + 1) * IR_SB, IR_SB)],
                    dstv[1 - bi], sem_i[1 - bi])
            if not (first and m == 0):
                drain_rows(sem_g[1 - p])
                if m == 0:
                    fire_scatter(1 - bi, SB_BLOCKS - 1, 1 - p)
                else:
                    fire_scatter(bi, m - 1, 1 - p)

    base = s * SB_TILE
    pltpu.sync_copy(src_hbm.at[c, pl.ds(base * IR_SB, IR_SB)], srcv0)
    pltpu.sync_copy(dst_hbm.at[pl.ds(base * IR_SB, IR_SB)], dstv0)
    process_sb(base, 0, first=True)
    process_sb(base + 1, 1)

    def _pair(qq, carry):
        process_sb(base + 2 * qq, 0)
        process_sb(base + 2 * qq + 1, 1)
        return carry
    lax.fori_loop(1, SB_TILE // 2, _pair, 0)

    drain_rows(sem_s[0])
    drain_rows(sem_g[1])
    fire_scatter(1, SB_BLOCKS - 1, 1)
    drain_rows(sem_s[1])
    drain_idx(sem_i[0])
    plsc.subcore_barrier()

    for k in range(ROWS_TILE // ZROWS):
        r0 = base_r + k * ZROWS
        pltpu.sync_copy(agg_sp.at[pl.ds(r0, ZROWS)], zbuf)
        pltpu.sync_copy(zbuf, out_hbm.at[pl.ds(r0, ZROWS), c])


def _make_sc_call():
    mesh = plsc.VectorSubcoreMesh(core_axis_name="c", subcore_axis_name="s")
    return pl.kernel(
        _sc_body,
        out_type=jax.ShapeDtypeStruct((TROWS, 2, 32), jnp.float32),
        mesh=mesh,
        scratch_types=[
            pltpu.VMEM_SHARED((N, 32), jnp.float32),
            pltpu.VMEM((ZROWS, 32), jnp.float32),
            pltpu.VMEM((SUB, EDGE_CHUNK, 32), jnp.float32),
            pltpu.VMEM((SUB, EDGE_CHUNK, 32), jnp.float32),
            pltpu.VMEM((IR_SB, EDGE_CHUNK), jnp.int32),
            pltpu.VMEM((IR_SB, EDGE_CHUNK), jnp.int32),
            pltpu.VMEM((IR_SB, EDGE_CHUNK), jnp.int32),
            pltpu.VMEM((IR_SB, EDGE_CHUNK), jnp.int32),
            pltpu.SemaphoreType.DMA,
            pltpu.SemaphoreType.DMA,
            pltpu.SemaphoreType.DMA,
            pltpu.SemaphoreType.DMA,
            pltpu.SemaphoreType.DMA,
            pltpu.SemaphoreType.DMA,
        ],
        compiler_params=pltpu.CompilerParams(use_tc_tiling_on_sc=False),
    )


def _stage_b_body(x_in, w2_ref, w1_ref, out_ref):
    h2 = jnp.maximum(x_in[...] @ _bd2(w2_ref[...]), 0.0)
    out_ref[...] = jnp.maximum(h2 @ _bd2(w1_ref[...]), 0.0)


def _stage_b(aggx, W2_0, W1_1):
    return pl.pallas_call(
        _stage_b_body,
        grid=(TX // BX,),
        in_specs=[
            pl.BlockSpec((BX, 128), lambda i: (i, 0)),
            pl.BlockSpec((INTER, HF), lambda i: (0, 0)),
            pl.BlockSpec((HF, INTER), lambda i: (0, 0)),
        ],
        out_specs=pl.BlockSpec((BX, 128), lambda i: (i, 0)),
        out_shape=jax.ShapeDtypeStruct((TX, 128), jnp.float32),
    )(aggx, W2_0, W1_1)


def _stage_c_body(x_in, pe_ref, po_ref, w2_ref, dev_ref, wq_ref, wk_ref,
                  wv_ref, wo_ref, ln1g_ref, ln1b_ref, fw1_ref, fb1_ref,
                  fw2_ref, fb2_ref, ln2g_ref, ln2b_ref, out_ref, asum_ref):
    xq2 = x_in[...].reshape(BB, G // 2, 128)[:, :Q // 2, :]
    xq2 = xq2.reshape(BB * Q // 2, 128)
    hq2 = jnp.maximum(xq2 @ _bd2(w2_ref[...]), 0.0)

    io_r = lax.broadcasted_iota(jnp.int32, (BB, R, Q // 2), 1)
    x_rep = jnp.zeros((BB, R, HF), jnp.float32)
    for a, p_ref in ((0, pe_ref), (1, po_ref)):
        mask = (p_ref[...][:, None, :] == io_r).astype(jnp.float32)
        hq_a = hq2[:, 64 * a:64 * a + 64].reshape(BB, Q // 2, HF)
        x_rep = x_rep + lax.dot_general(
            mask, hq_a, (((2,), (1,)), ((0,), (0,))))

    dev = jnp.broadcast_to(dev_ref[...][None], (BB, R, DEV))
    x = jnp.concatenate([x_rep, dev], axis=2)
    xf = x.reshape(BB * R, DM)

    q = (xf @ wq_ref[...]) * (1.0 / (DK ** 0.5))
    k = xf @ wk_ref[...]
    v = xf @ wv_ref[...]

    ctxs = []
    asum = jnp.zeros((BB, R, R), jnp.float32)
    for h in range(NH):
        qh = q[:, h * DK:(h + 1) * DK].reshape(BB, R, DK)
        kh = k[:, h * DK:(h + 1) * DK].reshape(BB, R, DK)
        vh = v[:, h * DV:(h + 1) * DV].reshape(BB, R, DV)
        s = lax.dot_general(qh, kh, (((2,), (2,)), ((0,), (0,))))
        s = s - jnp.max(s, axis=2, keepdims=True)
        es = jnp.exp(s)
        attn = es / jnp.sum(es, axis=2, keepdims=True)
        ctx = lax.dot_general(attn, vh, (((2,), (1,)), ((0,), (0,))))
        ctxs.append(ctx.reshape(BB * R, DV))
        asum = asum + attn
    mha = jnp.concatenate(ctxs, axis=1) @ wo_ref[...]

    def _ln(t, g, b):
        m = jnp.mean(t, axis=1, keepdims=True)
        d = t - m
        var = jnp.mean(d * d, axis=1, keepdims=True)
        return d * lax.rsqrt(var + 1e-6) * g + b

    res = _ln(xf + mha, ln1g_ref[...], ln1b_ref[...])
    ffn = jnp.maximum(res @ fw1_ref[...] + fb1_ref[...], 0.0)
    ffn = ffn @ fw2_ref[...] + fb2_ref[...]
    out = _ln(res + ffn, ln2g_ref[...], ln2b_ref[...])

    out_ref[...] = out.reshape(BB, R, DM)
    asum_ref[...] = asum


def _stage_c(aggx, phys_e, phys_o, W2_1, dev_table, Wq, Wk, Wv, Wo,
             ln1_g, ln1_b, ffn_W1, ffn_b1, ffn_W2, ffn_b2, ln2_g, ln2_b):
    full = lambda shape: pl.BlockSpec(shape, lambda i: tuple(0 for _ in shape))
    return pl.pallas_call(
        _stage_c_body,
        grid=(TX // CX,),
        in_specs=[
            pl.BlockSpec((CX, 128), lambda i: (i, 0)),
            pl.BlockSpec((BB, Q // 2), lambda i: (i, 0)),
            pl.BlockSpec((BB, Q // 2), lambda i: (i, 0)),
            full((INTER, HF)),
            full((R, DEV)),
            full((DM, NH * DK)),
            full((DM, NH * DK)),
            full((DM, NH * DV)),
            full((NH * DV, DM)),
            full((1, DM)), full((1, DM)),
            full((DM, DI)), full((1, DI)),
            full((DI, DM)), full((1, DM)),
            full((1, DM)), full((1, DM)),
        ],
        out_specs=[
            pl.BlockSpec((BB, R, DM), lambda i: (i, 0, 0)),
            pl.BlockSpec((BB, R, R), lambda i: (i, 0, 0)),
        ],
        out_shape=[
            jax.ShapeDtypeStruct((B, R, DM), jnp.float32),
            jax.ShapeDtypeStruct((B, R, R), jnp.float32),
        ],
    )(aggx, phys_e, phys_o, W2_1, dev_table, Wq, Wk, Wv, Wo,
      ln1_g.reshape(1, DM), ln1_b.reshape(1, DM),
      ffn_W1, ffn_b1.reshape(1, DI), ffn_W2, ffn_b2.reshape(1, DM),
      ln2_g.reshape(1, DM), ln2_b.reshape(1, DM))


def kernel(gate_types, edge_index, physical_idx, gate_embed, W1_0, W2_0,
           W1_1, W2_1, dev_table, Wq, Wk, Wv, Wo, ln1_g, ln1_b, ffn_W1,
           ffn_b1, ffn_W2, ffn_b2, ln2_g, ln2_b):
    src_b, dst_b = _prep(edge_index)

    sc_call = _make_sc_call()

    t0 = _stage_a(gate_types, gate_embed, W1_0)
    agg1 = sc_call(t0.reshape(2 * TROWS, 32), src_b, dst_b)
    t1 = _stage_b(agg1.reshape(TX, 128), W2_0, W1_1)
    agg2 = sc_call(t1.reshape(2 * TROWS, 32), src_b, dst_b)

    phys = physical_idx.astype(jnp.int32)
    phys_e = phys[:, 0:Q:2]
    phys_o = phys[:, 1:Q:2]
    outp, asum = _stage_c(agg2.reshape(TX, 128), phys_e, phys_o, W2_1,
                          dev_table, Wq, Wk, Wv, Wo, ln1_g, ln1_b, ffn_W1,
                          ffn_b1, ffn_W2, ffn_b2, ln2_g, ln2_b)
    return outp, asum
